# Initial kernel scaffold; baseline (speedup 1.0000x reference)
import jax
import jax.numpy as jnp
from jax.experimental import pallas as pl


def kernel(x, edge_index, Wo0, bo0, Wr0, Wo1, bo1, Wr1, Wo2, bo2, Wr2, Wo3, bo3, Wr3, Wo4, bo4, Wr4, Wo5, bo5, Wr5, Wo6, bo6, Wr6, Wo7, bo7, Wr7):
    raise NotImplementedError("write your pallas kernel here")


# speedup vs baseline: 11.8132x; 11.8132x over previous
import functools

import jax
import jax.numpy as jnp
from jax import lax
from jax.experimental import pallas as pl
from jax.experimental.pallas import tpu as pltpu
from jax.experimental.pallas import tpu_sc as plsc

N = 100000
E = 1600000
WIN = 128
NB = 4
WPT = 392
NGRP = WPT // NB
NTILES = 32
EP = NTILES * WPT * WIN
EROWS = EP // WIN
ROWS_PT = 6272
NP = 16 * ROWS_PT
M = N // 8
MP = NP // 8
BM = 256
GRID = (M + BM - 1) // BM


def _make_scatter_kernel(p, with_counts):
  mesh = plsc.VectorSubcoreMesh(core_axis_name="c", subcore_axis_name="s",
                                num_cores=2, num_subcores=16)
  t_type = jax.ShapeDtypeStruct((2, p, NP, 16), jnp.float32)
  if with_counts:
    out_type = [t_type, jax.ShapeDtypeStruct((2, 2, NP, 16), jnp.float32)]
  else:
    out_type = t_type

  @functools.partial(
      pl.kernel,
      out_type=out_type,
      mesh=mesh,
      compiler_params=pltpu.CompilerParams(use_tc_tiling_on_sc=False),
      scratch_types=[
          pltpu.VMEM((2 * NB, WIN), jnp.int32),
          pltpu.VMEM((2 * NB, WIN), jnp.int32),
          pltpu.VMEM((2 * NB, WIN), jnp.int32),
          pltpu.VMEM((2 * NB, WIN, 16), jnp.float32),
          pltpu.VMEM((WIN, 16), jnp.float32),
          pltpu.VMEM((WIN, 16), jnp.float32),
          pltpu.VMEM_SHARED((NP, 16), jnp.float32),
          pltpu.SemaphoreType.DMA((NB,)),
          pltpu.SemaphoreType.DMA,
      ],
  )
  def scatter_kernel(table, src2, dst2, zeros_hbm, ones_hbm, *rest):
    if with_counts:
      out, cnt_out = rest[0], rest[1]
      scratch = rest[2:]
    else:
      out = rest[0]
      cnt_out = None
      scratch = rest[1:]
    sbuf, dbuf, selbuf, rows, zeros_v, ones_v, accum, sems, isem = scratch
    c = lax.axis_index("c")
    s = lax.axis_index("s")
    erow0 = (c * 16 + s) * WPT
    r0 = s * ROWS_PT
    pltpu.sync_copy(zeros_hbm, zeros_v)
    if with_counts:
      pltpu.sync_copy(ones_hbm, ones_v)

    def zero_accum():
      for k in range(ROWS_PT // WIN):
        pltpu.sync_copy(zeros_v, accum.at[pl.ds(r0 + k * WIN, WIN)])

    def flush(dst_ref):
      sl = pl.ds(r0, ROWS_PT)
      pltpu.sync_copy(accum.at[sl], dst_ref.at[c, sl])

    def load_group(g, par, want_src):
      base = erow0 + g * NB
      if want_src:
        pltpu.sync_copy(src2.at[pl.ds(base, NB)],
                        sbuf.at[pl.ds(par * NB, NB)])
      pltpu.sync_copy(dst2.at[pl.ds(base, NB)],
                      dbuf.at[pl.ds(par * NB, NB)])

    for j in range(p):
      tbl = table.at[j]
      zero_accum()
      plsc.subcore_barrier()

      load_group(0, 0, True)
      for b in range(NB):
        pltpu.async_copy(tbl.at[sbuf.at[b]], rows.at[b], sems.at[b])

      def one_group(g, par):
        nxt = 1 - par

        @pl.when(g + 1 < NGRP)
        def _():
          load_group(g + 1, nxt, True)

        for b in range(NB):
          i = par * NB + b
          pltpu.make_async_copy(tbl.at[sbuf.at[i]], rows.at[i],
                                sems.at[b]).wait()
          pltpu.sync_copy(rows.at[i], accum.at[dbuf.at[i]], add=True)

          @pl.when(g + 1 < NGRP)
          def _():
            ni = nxt * NB + b
            pltpu.async_copy(tbl.at[sbuf.at[ni]], rows.at[ni], sems.at[b])

      def body(i, carry):
        one_group(2 * i, 0)
        one_group(2 * i + 1, 1)
        return carry

      lax.fori_loop(0, NGRP // 2, body, 0)
      plsc.subcore_barrier()
      flush(out.at[slice(None), j])

    if with_counts:
      zero_accum()
      plsc.subcore_barrier()
      load_group(0, 0, False)

      def deg_group(g, par):
        @pl.when(g + 1 < NGRP)
        def _():
          load_group(g + 1, 1 - par, False)

        for b in range(NB):
          pltpu.sync_copy(ones_v, accum.at[dbuf.at[par * NB + b]], add=True)

      def deg_body(i, carry):
        deg_group(2 * i, 0)
        deg_group(2 * i + 1, 1)
        return carry

      lax.fori_loop(0, NGRP // 2, deg_body, 0)
      plsc.subcore_barrier()
      flush(cnt_out.at[slice(None), 0])

      zero_accum()
      plsc.subcore_barrier()
      trash = jnp.full((16,), N, dtype=jnp.int32)
      load_group(0, 0, True)

      def self_group(g, par):
        @pl.when(g + 1 < NGRP)
        def _():
          load_group(g + 1, 1 - par, True)

        for b in range(NB):
          i = par * NB + b
          for k in range(WIN // 16):
            sv = sbuf[i, pl.ds(k * 16, 16)]
            dv = dbuf[i, pl.ds(k * 16, 16)]
            selbuf[i, pl.ds(k * 16, 16)] = jnp.where(sv == dv, dv, trash)
          pltpu.sync_copy(ones_v, accum.at[selbuf.at[i]], add=True)

      def self_body(i, carry):
        self_group(2 * i, 0)
        self_group(2 * i + 1, 1)
        return carry

      lax.fori_loop(0, NGRP // 2, self_body, 0)
      plsc.subcore_barrier()
      flush(cnt_out.at[slice(None), 1])

  return scatter_kernel


def _lrelu(v):
  return jnp.where(v >= 0.0, v, 0.1 * v)


def _mm(a, w):
  return jax.lax.dot_general(a, w, (((1,), (0,)), ((), ())),
                             precision=jax.lax.Precision.HIGHEST,
                             preferred_element_type=jnp.float32)


def _pk_spec(p):
  return pl.BlockSpec((p, BM, 128), lambda i: (0, i, 0))


_T_SPEC = lambda p: pl.BlockSpec((2, p, BM, 128), lambda i: (0, 0, i, 0))
_CNT_SPEC = pl.BlockSpec((2, 2, BM, 128), lambda i: (0, 0, i, 0))
_AB_SPEC = pl.BlockSpec((2, BM, 128), lambda i: (0, i, 0))


def _full_spec(shape):
  nd = len(shape)
  return pl.BlockSpec(shape, lambda i: (0,) * nd)


def _scalars_from_cnt(cnt):
  d = cnt[0, 0] + cnt[1, 0]
  s = cnt[0, 1] + cnt[1, 1]
  deg = 1.0 + d - s
  a = 1.0 / jnp.maximum(deg, 1.0)
  return a, a * (1.0 - s)


def _make_dense(p_in, p_out, form_b, p_next=None, first=False):
  def body(*refs):
    it = iter(refs)
    h_ref = next(it)
    g_ref = next(it) if form_b else None
    t_ref = next(it)
    sc_ref = next(it)
    wok_ref = None if form_b else next(it)
    bop_ref = next(it)
    wrk_ref = next(it)
    wnk_ref = next(it) if p_next is not None else None
    out_ref = next(it)
    gout_ref = next(it) if p_next is not None else None
    ab_ref = next(it) if first else None

    if first:
      a, b = _scalars_from_cnt(sc_ref[...])
      ab_ref[0] = a
      ab_ref[1] = b
    else:
      a = sc_ref[0]
      b = sc_ref[1]

    hs = [h_ref[i] for i in range(p_in)]
    vals = []
    for po in range(p_out):
      if form_b:
        val = a * (t_ref[0, po] + t_ref[1, po]) + b * g_ref[po]
        for pi in range(p_in):
          val = val + _mm(hs[pi], wrk_ref[pi, po])
      else:
        val = None
        for pi in range(p_in):
          agg = a * (t_ref[0, pi] + t_ref[1, pi]) + b * hs[pi]
          term = _mm(agg, wok_ref[pi, po]) + _mm(hs[pi], wrk_ref[pi, po])
          val = term if val is None else val + term
      val = _lrelu(val + bop_ref[po])
      vals.append(val)
      out_ref[po] = val
    if p_next is not None:
      for pn in range(p_next):
        gv = None
        for po in range(p_out):
          term = _mm(vals[po], wnk_ref[po, pn])
          gv = term if gv is None else gv + term
        gout_ref[pn] = gv

  in_specs = [_pk_spec(p_in)]
  if form_b:
    in_specs.append(_pk_spec(p_out))
  in_specs.append(_T_SPEC(p_out if form_b else p_in))
  in_specs.append(_CNT_SPEC if first else _AB_SPEC)
  if not form_b:
    in_specs.append(_full_spec((p_in, p_out, 128, 128)))
  in_specs.append(_full_spec((p_out, 128)))
  in_specs.append(_full_spec((p_in, p_out, 128, 128)))
  if p_next is not None:
    in_specs.append(_full_spec((p_out, p_next, 128, 128)))

  out_shape = [jax.ShapeDtypeStruct((p_out, M, 128), jnp.float32)]
  out_specs = [_pk_spec(p_out)]
  if p_next is not None:
    out_shape.append(jax.ShapeDtypeStruct((p_next, M, 128), jnp.float32))
    out_specs.append(_pk_spec(p_next))
  if first:
    out_shape.append(jax.ShapeDtypeStruct((2, MP, 128), jnp.float32))
    out_specs.append(_AB_SPEC)

  return pl.pallas_call(
      body,
      grid=(GRID,),
      in_specs=in_specs,
      out_specs=out_specs if len(out_specs) > 1 else out_specs[0],
      out_shape=out_shape if len(out_shape) > 1 else out_shape[0],
  )


_scatter1c = _make_scatter_kernel(1, True)
_scatter1 = _make_scatter_kernel(1, False)
_scatter2 = _make_scatter_kernel(2, False)
_scatter4 = _make_scatter_kernel(4, False)

_dense0 = _make_dense(1, 1, False, first=True)
_dense1 = _make_dense(1, 1, False)
_dense2 = _make_dense(1, 2, False)
_dense3 = _make_dense(2, 4, False)
_dense4 = _make_dense(4, 4, False, p_next=2)
_dense5 = _make_dense(4, 2, True, p_next=1)
_dense6 = _make_dense(2, 1, True)
_dense7 = _make_dense(1, 1, False)


def _pad16(w):
  di, do = w.shape
  return jnp.pad(w, ((0, (-di) % 16), (0, (-do) % 16)))


def _kron8(w):
  w = _pad16(w)
  pi, po = w.shape[0] // 16, w.shape[1] // 16
  i8 = jnp.eye(8, dtype=w.dtype)
  return jnp.stack([
      jnp.stack([jnp.kron(i8, w[16 * i:16 * i + 16, 16 * j:16 * j + 16])
                 for j in range(po)])
      for i in range(pi)])


def _packb(bo):
  bo = jnp.pad(bo, (0, (-bo.shape[0]) % 16))
  po = bo.shape[0] // 16
  return jnp.tile(bo.reshape(po, 1, 16), (1, 8, 1)).reshape(po, 128)


def _sc_view(hpk):
  p = hpk.shape[0]
  return hpk.reshape(p, M * 8, 16)


def kernel(x, edge_index,
           Wo0, bo0, Wr0, Wo1, bo1, Wr1, Wo2, bo2, Wr2, Wo3, bo3, Wr3,
           Wo4, bo4, Wr4, Wo5, bo5, Wr5, Wo6, bo6, Wr6, Wo7, bo7, Wr7):
  f32 = jnp.float32
  src = jnp.concatenate(
      [edge_index[0], jnp.zeros((EP - E,), jnp.int32)]).reshape(EROWS, WIN)
  dst = jnp.concatenate(
      [edge_index[1], jnp.full((EP - E,), N, jnp.int32)]).reshape(EROWS, WIN)
  zeros16 = jnp.zeros((WIN, 16), f32)
  ones16 = jnp.ones((WIN, 16), f32)

  xpk = jnp.pad(x.reshape(M, 8, 3), ((0, 0), (0, 0), (0, 13))).reshape(
      1, M, 128)

  wok = [_kron8(w) for w in (Wo0, Wo1, Wo2, Wo3, Wo4, Wo5, Wo6, Wo7)]
  wrk = [_kron8(w) for w in (Wr0, Wr1, Wr2, Wr3, Wr4, Wr5, Wr6, Wr7)]
  bop = [_packb(b) for b in (bo0, bo1, bo2, bo3, bo4, bo5, bo6, bo7)]

  def t_view(t):
    return t.reshape(t.shape[0], t.shape[1], MP, 128)

  t0, cnt = _scatter1c(_sc_view(xpk), src, dst, zeros16, ones16)
  cntv = cnt.reshape(2, 2, MP, 128)
  h1, ab = _dense0(xpk, t_view(t0), cntv, wok[0], bop[0], wrk[0])
  t1 = _scatter1(_sc_view(h1), src, dst, zeros16, ones16)
  h2 = _dense1(h1, t_view(t1), ab, wok[1], bop[1], wrk[1])
  t2 = _scatter1(_sc_view(h2), src, dst, zeros16, ones16)
  h3 = _dense2(h2, t_view(t2), ab, wok[2], bop[2], wrk[2])
  t3 = _scatter2(_sc_view(h3), src, dst, zeros16, ones16)
  h4 = _dense3(h3, t_view(t3), ab, wok[3], bop[3], wrk[3])
  t4 = _scatter4(_sc_view(h4), src, dst, zeros16, ones16)
  h5, g5 = _dense4(h4, t_view(t4), ab, wok[4], bop[4], wrk[4], wok[5])
  t5 = _scatter2(_sc_view(g5), src, dst, zeros16, ones16)
  h6, g6 = _dense5(h5, g5, t_view(t5), ab, bop[5], wrk[5], wok[6])
  t6 = _scatter1(_sc_view(g6), src, dst, zeros16, ones16)
  h7 = _dense6(h6, g6, t_view(t6), ab, bop[6], wrk[6])
  t7 = _scatter1(_sc_view(h7), src, dst, zeros16, ones16)
  out8 = _dense7(h7, t_view(t7), ab, wok[7], bop[7], wrk[7])
  return out8.reshape(M, 8, 16)[:, :, 0].reshape(-1)


# speedup vs baseline: 15.7980x; 1.3373x over previous
import functools

import jax
import jax.numpy as jnp
from jax import lax
from jax.experimental import pallas as pl
from jax.experimental.pallas import tpu as pltpu
from jax.experimental.pallas import tpu_sc as plsc

N = 100000
E = 1600000
WIN = 128
NB = 4
WPT = 392
NGRP = WPT // NB
NTILES = 32
EP = NTILES * WPT * WIN
EROWS = EP // WIN
ROWS_PT = 6272
NP = 16 * ROWS_PT
M = N // 8
MP = NP // 8
BM = 256
GRID = (M + BM - 1) // BM


def _make_scatter_kernel(p, with_counts):
  mesh = plsc.VectorSubcoreMesh(core_axis_name="c", subcore_axis_name="s",
                                num_cores=2, num_subcores=16)
  t_type = jax.ShapeDtypeStruct((2, p, NP, 16), jnp.float32)
  if with_counts:
    out_type = [t_type, jax.ShapeDtypeStruct((2, 2, NP, 16), jnp.float32)]
  else:
    out_type = t_type

  @functools.partial(
      pl.kernel,
      out_type=out_type,
      mesh=mesh,
      compiler_params=pltpu.CompilerParams(use_tc_tiling_on_sc=False),
      scratch_types=[
          pltpu.VMEM((2 * NB, WIN), jnp.int32),
          pltpu.VMEM((2 * NB, WIN), jnp.int32),
          pltpu.VMEM((2 * NB, WIN), jnp.int32),
          pltpu.VMEM((2 * NB, WIN, 16), jnp.float32),
          pltpu.VMEM((WIN, 16), jnp.float32),
          pltpu.VMEM((WIN, 16), jnp.float32),
          pltpu.VMEM_SHARED((NP, 16), jnp.float32),
          pltpu.SemaphoreType.DMA((NB,)),
          pltpu.SemaphoreType.DMA,
      ],
  )
  def scatter_kernel(table, src2, dst2, zeros_hbm, ones_hbm, *rest):
    if with_counts:
      out, cnt_out = rest[0], rest[1]
      scratch = rest[2:]
    else:
      out = rest[0]
      cnt_out = None
      scratch = rest[1:]
    sbuf, dbuf, selbuf, rows, zeros_v, ones_v, accum, sems, isem = scratch
    c = lax.axis_index("c")
    s = lax.axis_index("s")
    erow0 = (c * 16 + s) * WPT
    r0 = s * ROWS_PT
    pltpu.sync_copy(zeros_hbm, zeros_v)
    if with_counts:
      pltpu.sync_copy(ones_hbm, ones_v)

    def zero_accum():
      for k in range(ROWS_PT // WIN):
        pltpu.sync_copy(zeros_v, accum.at[pl.ds(r0 + k * WIN, WIN)])

    def flush(dst_ref):
      sl = pl.ds(r0, ROWS_PT)
      pltpu.sync_copy(accum.at[sl], dst_ref.at[c, sl])

    def load_group(g, par, want_src):
      base = erow0 + g * NB
      if want_src:
        pltpu.sync_copy(src2.at[pl.ds(base, NB)],
                        sbuf.at[pl.ds(par * NB, NB)])
      pltpu.sync_copy(dst2.at[pl.ds(base, NB)],
                      dbuf.at[pl.ds(par * NB, NB)])

    for j in range(p):
      tbl = table.at[j]
      zero_accum()
      plsc.subcore_barrier()

      load_group(0, 0, True)
      for b in range(NB):
        pltpu.async_copy(tbl.at[sbuf.at[b]], rows.at[b], sems.at[b])

      def one_group(g, par):
        nxt = 1 - par

        @pl.when(g + 1 < NGRP)
        def _():
          load_group(g + 1, nxt, True)

        for b in range(NB):
          i = par * NB + b
          pltpu.make_async_copy(tbl.at[sbuf.at[i]], rows.at[i],
                                sems.at[b]).wait()
          pltpu.sync_copy(rows.at[i], accum.at[dbuf.at[i]], add=True)

          @pl.when(g + 1 < NGRP)
          def _():
            ni = nxt * NB + b
            pltpu.async_copy(tbl.at[sbuf.at[ni]], rows.at[ni], sems.at[b])

      def body(i, carry):
        one_group(2 * i, 0)
        one_group(2 * i + 1, 1)
        return carry

      lax.fori_loop(0, NGRP // 2, body, 0)
      plsc.subcore_barrier()
      flush(out.at[slice(None), j])

    if with_counts:
      zero_accum()
      plsc.subcore_barrier()
      load_group(0, 0, False)

      def deg_group(g, par):
        @pl.when(g + 1 < NGRP)
        def _():
          load_group(g + 1, 1 - par, False)

        for b in range(NB):
          pltpu.sync_copy(ones_v, accum.at[dbuf.at[par * NB + b]], add=True)

      def deg_body(i, carry):
        deg_group(2 * i, 0)
        deg_group(2 * i + 1, 1)
        return carry

      lax.fori_loop(0, NGRP // 2, deg_body, 0)
      plsc.subcore_barrier()
      flush(cnt_out.at[slice(None), 0])

      zero_accum()
      plsc.subcore_barrier()
      trash = jnp.full((16,), N, dtype=jnp.int32)
      load_group(0, 0, True)

      def self_group(g, par):
        @pl.when(g + 1 < NGRP)
        def _():
          load_group(g + 1, 1 - par, True)

        for b in range(NB):
          i = par * NB + b
          for k in range(WIN // 16):
            sv = sbuf[i, pl.ds(k * 16, 16)]
            dv = dbuf[i, pl.ds(k * 16, 16)]
            selbuf[i, pl.ds(k * 16, 16)] = jnp.where(sv == dv, dv, trash)
          pltpu.sync_copy(ones_v, accum.at[selbuf.at[i]], add=True)

      def self_body(i, carry):
        self_group(2 * i, 0)
        self_group(2 * i + 1, 1)
        return carry

      lax.fori_loop(0, NGRP // 2, self_body, 0)
      plsc.subcore_barrier()
      flush(cnt_out.at[slice(None), 1])

  return scatter_kernel


def _lrelu(v):
  return jnp.where(v >= 0.0, v, 0.1 * v)


def _mm(a, w):
  return jax.lax.dot_general(a, w, (((1,), (0,)), ((), ())),
                             precision=jax.lax.Precision.HIGHEST,
                             preferred_element_type=jnp.float32)


def _pk_spec(p):
  return pl.BlockSpec((p, BM, 128), lambda i: (0, i, 0))


_T_SPEC = lambda p: pl.BlockSpec((2, p, BM, 128), lambda i: (0, 0, i, 0))
_CNT_SPEC = pl.BlockSpec((2, 2, BM, 128), lambda i: (0, 0, i, 0))
_AB_SPEC = pl.BlockSpec((2, BM, 128), lambda i: (0, i, 0))


def _full_spec(shape):
  nd = len(shape)
  return pl.BlockSpec(shape, lambda i: (0,) * nd)


def _scalars_from_cnt(cnt):
  d = cnt[0, 0] + cnt[1, 0]
  s = cnt[0, 1] + cnt[1, 1]
  deg = 1.0 + d - s
  a = 1.0 / jnp.maximum(deg, 1.0)
  return a, a * (1.0 - s)


def _make_dense(p_in, p_out, form_b, p_next=None, first=False):
  def body(*refs):
    it = iter(refs)
    h_ref = next(it)
    g_ref = next(it) if form_b else None
    t_ref = next(it)
    sc_ref = next(it)
    wok_ref = None if form_b else next(it)
    bop_ref = next(it)
    wrk_ref = next(it)
    wnk_ref = next(it) if p_next is not None else None
    out_ref = next(it)
    gout_ref = next(it) if p_next is not None else None
    ab_ref = next(it) if first else None

    if first:
      a, b = _scalars_from_cnt(sc_ref[...])
      ab_ref[0] = a
      ab_ref[1] = b
    else:
      a = sc_ref[0]
      b = sc_ref[1]

    hs = [h_ref[i] for i in range(p_in)]
    vals = []
    for po in range(p_out):
      if form_b:
        val = a * (t_ref[0, po] + t_ref[1, po]) + b * g_ref[po]
        for pi in range(p_in):
          val = val + _mm(hs[pi], wrk_ref[pi, po])
      else:
        val = None
        for pi in range(p_in):
          agg = a * (t_ref[0, pi] + t_ref[1, pi]) + b * hs[pi]
          term = _mm(agg, wok_ref[pi, po]) + _mm(hs[pi], wrk_ref[pi, po])
          val = term if val is None else val + term
      val = _lrelu(val + bop_ref[po])
      vals.append(val)
      out_ref[po] = val
    if p_next is not None:
      for pn in range(p_next):
        gv = None
        for po in range(p_out):
          term = _mm(vals[po], wnk_ref[po, pn])
          gv = term if gv is None else gv + term
        gout_ref[pn] = gv

  in_specs = [_pk_spec(p_in)]
  if form_b:
    in_specs.append(_pk_spec(p_out))
  in_specs.append(_T_SPEC(p_out if form_b else p_in))
  in_specs.append(_CNT_SPEC if first else _AB_SPEC)
  if not form_b:
    in_specs.append(_full_spec((p_in, p_out, 128, 128)))
  in_specs.append(_full_spec((p_out, 128)))
  in_specs.append(_full_spec((p_in, p_out, 128, 128)))
  if p_next is not None:
    in_specs.append(_full_spec((p_out, p_next, 128, 128)))

  out_shape = [jax.ShapeDtypeStruct((p_out, M, 128), jnp.float32)]
  out_specs = [_pk_spec(p_out)]
  if p_next is not None:
    out_shape.append(jax.ShapeDtypeStruct((p_next, M, 128), jnp.float32))
    out_specs.append(_pk_spec(p_next))
  if first:
    out_shape.append(jax.ShapeDtypeStruct((2, MP, 128), jnp.float32))
    out_specs.append(_AB_SPEC)

  return pl.pallas_call(
      body,
      grid=(GRID,),
      in_specs=in_specs,
      out_specs=out_specs if len(out_specs) > 1 else out_specs[0],
      out_shape=out_shape if len(out_shape) > 1 else out_shape[0],
  )


def _xpack_body(x_ref, p_ref, o_ref):
  o_ref[0] = _mm(x_ref[...], p_ref[...])


_xpack = pl.pallas_call(
    _xpack_body,
    grid=(GRID,),
    in_specs=[pl.BlockSpec((BM, 24), lambda i: (i, 0)),
              _full_spec((24, 128))],
    out_specs=pl.BlockSpec((1, BM, 128), lambda i: (0, i, 0)),
    out_shape=jax.ShapeDtypeStruct((1, M, 128), jnp.float32),
)


def _unpack_body(v_ref, s_ref, o_ref):
  o_ref[...] = _mm(v_ref[0], s_ref[...])


_unpack = pl.pallas_call(
    _unpack_body,
    grid=(GRID,),
    in_specs=[pl.BlockSpec((1, BM, 128), lambda i: (0, i, 0)),
              _full_spec((128, 8))],
    out_specs=pl.BlockSpec((BM, 8), lambda i: (i, 0)),
    out_shape=jax.ShapeDtypeStruct((M, 8), jnp.float32),
)


_scatter1c = _make_scatter_kernel(1, True)
_scatter1 = _make_scatter_kernel(1, False)
_scatter2 = _make_scatter_kernel(2, False)
_scatter4 = _make_scatter_kernel(4, False)

_dense0 = _make_dense(1, 1, False, first=True)
_dense1 = _make_dense(1, 1, False)
_dense2 = _make_dense(1, 2, False)
_dense3 = _make_dense(2, 4, False)
_dense4 = _make_dense(4, 4, False, p_next=2)
_dense5 = _make_dense(4, 2, True, p_next=1)
_dense6 = _make_dense(2, 1, True)
_dense7 = _make_dense(1, 1, False)


def _pad16(w):
  di, do = w.shape
  return jnp.pad(w, ((0, (-di) % 16), (0, (-do) % 16)))


def _kron8(w):
  w = _pad16(w)
  pi, po = w.shape[0] // 16, w.shape[1] // 16
  i8 = jnp.eye(8, dtype=w.dtype)
  return jnp.stack([
      jnp.stack([jnp.kron(i8, w[16 * i:16 * i + 16, 16 * j:16 * j + 16])
                 for j in range(po)])
      for i in range(pi)])


def _packb(bo):
  bo = jnp.pad(bo, (0, (-bo.shape[0]) % 16))
  po = bo.shape[0] // 16
  return jnp.tile(bo.reshape(po, 1, 16), (1, 8, 1)).reshape(po, 128)


def _sc_view(hpk):
  p = hpk.shape[0]
  return hpk.reshape(p, M * 8, 16)


def kernel(x, edge_index,
           Wo0, bo0, Wr0, Wo1, bo1, Wr1, Wo2, bo2, Wr2, Wo3, bo3, Wr3,
           Wo4, bo4, Wr4, Wo5, bo5, Wr5, Wo6, bo6, Wr6, Wo7, bo7, Wr7):
  f32 = jnp.float32
  pad = jnp.stack([jnp.zeros((EP - E,), jnp.int32),
                   jnp.full((EP - E,), N, jnp.int32)])
  ei3 = jnp.concatenate([edge_index, pad], axis=1).reshape(2, EROWS, WIN)
  src = ei3[0]
  dst = ei3[1]
  zeros16 = jnp.zeros((WIN, 16), f32)
  ones16 = jnp.ones((WIN, 16), f32)

  q = jnp.arange(8).repeat(3)
  f = jnp.tile(jnp.arange(3), 8)
  perm = jnp.zeros((24, 128), f32).at[3 * q + f, 16 * q + f].set(1.0)
  xpk = _xpack(x.reshape(M, 24), perm)

  wok = [_kron8(w) for w in (Wo0, Wo1, Wo2, Wo3, Wo4, Wo5, Wo6, Wo7)]
  wrk = [_kron8(w) for w in (Wr0, Wr1, Wr2, Wr3, Wr4, Wr5, Wr6, Wr7)]
  bop = [_packb(b) for b in (bo0, bo1, bo2, bo3, bo4, bo5, bo6, bo7)]

  def t_view(t):
    return t.reshape(t.shape[0], t.shape[1], MP, 128)

  t0, cnt = _scatter1c(_sc_view(xpk), src, dst, zeros16, ones16)
  cntv = cnt.reshape(2, 2, MP, 128)
  h1, ab = _dense0(xpk, t_view(t0), cntv, wok[0], bop[0], wrk[0])
  t1 = _scatter1(_sc_view(h1), src, dst, zeros16, ones16)
  h2 = _dense1(h1, t_view(t1), ab, wok[1], bop[1], wrk[1])
  t2 = _scatter1(_sc_view(h2), src, dst, zeros16, ones16)
  h3 = _dense2(h2, t_view(t2), ab, wok[2], bop[2], wrk[2])
  t3 = _scatter2(_sc_view(h3), src, dst, zeros16, ones16)
  h4 = _dense3(h3, t_view(t3), ab, wok[3], bop[3], wrk[3])
  t4 = _scatter4(_sc_view(h4), src, dst, zeros16, ones16)
  h5, g5 = _dense4(h4, t_view(t4), ab, wok[4], bop[4], wrk[4], wok[5])
  t5 = _scatter2(_sc_view(g5), src, dst, zeros16, ones16)
  h6, g6 = _dense5(h5, g5, t_view(t5), ab, bop[5], wrk[5], wok[6])
  t6 = _scatter1(_sc_view(g6), src, dst, zeros16, ones16)
  h7 = _dense6(h6, g6, t_view(t6), ab, bop[6], wrk[6])
  t7 = _scatter1(_sc_view(h7), src, dst, zeros16, ones16)
  out8 = _dense7(h7, t_view(t7), ab, wok[7], bop[7], wrk[7])
  sel = jnp.zeros((128, 8), f32).at[jnp.arange(8) * 16, jnp.arange(8)].set(1.0)
  return _unpack(out8, sel).reshape(N)


# speedup vs baseline: 16.6991x; 1.0570x over previous
import functools

import jax
import jax.numpy as jnp
from jax import lax
from jax.experimental import pallas as pl
from jax.experimental.pallas import tpu as pltpu
from jax.experimental.pallas import tpu_sc as plsc

N = 100000
E = 1600000
WIN = 128
NB = 4
WPT = 392
NGRP = WPT // NB
NTILES = 32
EP = NTILES * WPT * WIN
EROWS = EP // WIN
ROWS_PT = 6272
NP = 16 * ROWS_PT
M = N // 8
MP = NP // 8
BM = 256
GRID = (M + BM - 1) // BM


def _make_scatter_kernel(p, with_counts):
  mesh = plsc.VectorSubcoreMesh(core_axis_name="c", subcore_axis_name="s",
                                num_cores=2, num_subcores=16)
  t_type = jax.ShapeDtypeStruct((2, p, NP, 16), jnp.float32)
  if with_counts:
    out_type = [t_type, jax.ShapeDtypeStruct((2, 2, NP, 16), jnp.float32)]
  else:
    out_type = t_type

  @functools.partial(
      pl.kernel,
      out_type=out_type,
      mesh=mesh,
      compiler_params=pltpu.CompilerParams(use_tc_tiling_on_sc=False),
      scratch_types=[
          pltpu.VMEM((2 * NB, WIN), jnp.int32),
          pltpu.VMEM((2 * NB, WIN), jnp.int32),
          pltpu.VMEM((2 * NB, WIN), jnp.int32),
          pltpu.VMEM((2 * NB, WIN, 16), jnp.float32),
          pltpu.VMEM((WIN, 16), jnp.float32),
          pltpu.VMEM((WIN, 16), jnp.float32),
          pltpu.VMEM_SHARED((NP, 16), jnp.float32),
          pltpu.SemaphoreType.DMA((NB,)),
          pltpu.SemaphoreType.DMA((2 * NB,)),
      ],
  )
  def scatter_kernel(table, src2, dst2, zeros_hbm, ones_hbm, *rest):
    if with_counts:
      out, cnt_out = rest[0], rest[1]
      scratch = rest[2:]
    else:
      out = rest[0]
      cnt_out = None
      scratch = rest[1:]
    sbuf, dbuf, selbuf, rows, zeros_v, ones_v, accum, sems, ssems = scratch
    c = lax.axis_index("c")
    s = lax.axis_index("s")
    erow0 = (c * 16 + s) * WPT
    r0 = s * ROWS_PT
    pltpu.sync_copy(zeros_hbm, zeros_v)
    if with_counts:
      pltpu.sync_copy(ones_hbm, ones_v)

    def zero_accum():
      for k in range(ROWS_PT // WIN):
        pltpu.sync_copy(zeros_v, accum.at[pl.ds(r0 + k * WIN, WIN)])

    def flush(dst_ref):
      sl = pl.ds(r0, ROWS_PT)
      pltpu.sync_copy(accum.at[sl], dst_ref.at[c, sl])

    def load_group(g, par, want_src):
      base = erow0 + g * NB
      if want_src:
        pltpu.sync_copy(src2.at[pl.ds(base, NB)],
                        sbuf.at[pl.ds(par * NB, NB)])
      pltpu.sync_copy(dst2.at[pl.ds(base, NB)],
                      dbuf.at[pl.ds(par * NB, NB)])

    def drain_scatters(par):
      for b in range(NB):
        i = par * NB + b
        pltpu.make_async_copy(rows.at[i], accum.at[dbuf.at[i]],
                              ssems.at[i]).wait()

    for j in range(p):
      tbl = table.at[j]
      zero_accum()
      plsc.subcore_barrier()

      load_group(0, 0, True)
      for b in range(NB):
        pltpu.async_copy(tbl.at[sbuf.at[b]], rows.at[b], sems.at[b])

      def one_group(g, par):
        nxt = 1 - par

        @pl.when(g > 0)
        def _():
          drain_scatters(nxt)

        @pl.when(g + 1 < NGRP)
        def _():
          load_group(g + 1, nxt, True)

        for b in range(NB):
          i = par * NB + b
          pltpu.make_async_copy(tbl.at[sbuf.at[i]], rows.at[i],
                                sems.at[b]).wait()
          pltpu.async_copy(rows.at[i], accum.at[dbuf.at[i]], ssems.at[i],
                           add=True)

          @pl.when(g + 1 < NGRP)
          def _():
            ni = nxt * NB + b
            pltpu.async_copy(tbl.at[sbuf.at[ni]], rows.at[ni], sems.at[b])

      def body(i, carry):
        one_group(2 * i, 0)
        one_group(2 * i + 1, 1)
        return carry

      lax.fori_loop(0, NGRP // 2, body, 0)
      drain_scatters((NGRP - 1) % 2)
      plsc.subcore_barrier()
      flush(out.at[slice(None), j])

    if with_counts:
      def drain_ones(par, idx_buf):
        for b in range(NB):
          i = par * NB + b
          pltpu.make_async_copy(ones_v, accum.at[idx_buf.at[i]],
                                ssems.at[i]).wait()

      zero_accum()
      plsc.subcore_barrier()
      load_group(0, 0, False)

      def deg_group(g, par):
        nxt = 1 - par

        @pl.when(g > 0)
        def _():
          drain_ones(nxt, dbuf)

        @pl.when(g + 1 < NGRP)
        def _():
          load_group(g + 1, nxt, False)

        for b in range(NB):
          i = par * NB + b
          pltpu.async_copy(ones_v, accum.at[dbuf.at[i]], ssems.at[i],
                           add=True)

      def deg_body(i, carry):
        deg_group(2 * i, 0)
        deg_group(2 * i + 1, 1)
        return carry

      lax.fori_loop(0, NGRP // 2, deg_body, 0)
      drain_ones((NGRP - 1) % 2, dbuf)
      plsc.subcore_barrier()
      flush(cnt_out.at[slice(None), 0])

      zero_accum()
      plsc.subcore_barrier()
      trash = jnp.full((16,), N, dtype=jnp.int32)
      load_group(0, 0, True)

      def self_group(g, par):
        nxt = 1 - par

        @pl.when(g > 0)
        def _():
          drain_ones(nxt, selbuf)

        @pl.when(g + 1 < NGRP)
        def _():
          load_group(g + 1, nxt, True)

        for b in range(NB):
          i = par * NB + b
          for k in range(WIN // 16):
            sv = sbuf[i, pl.ds(k * 16, 16)]
            dv = dbuf[i, pl.ds(k * 16, 16)]
            selbuf[i, pl.ds(k * 16, 16)] = jnp.where(sv == dv, dv, trash)
          pltpu.async_copy(ones_v, accum.at[selbuf.at[i]], ssems.at[i],
                           add=True)

      def self_body(i, carry):
        self_group(2 * i, 0)
        self_group(2 * i + 1, 1)
        return carry

      lax.fori_loop(0, NGRP // 2, self_body, 0)
      drain_ones((NGRP - 1) % 2, selbuf)
      plsc.subcore_barrier()
      flush(cnt_out.at[slice(None), 1])

  return scatter_kernel


def _lrelu(v):
  return jnp.where(v >= 0.0, v, 0.1 * v)


def _mm(a, w):
  return jax.lax.dot_general(a, w, (((1,), (0,)), ((), ())),
                             precision=jax.lax.Precision.HIGHEST,
                             preferred_element_type=jnp.float32)


def _pk_spec(p):
  return pl.BlockSpec((p, BM, 128), lambda i: (0, i, 0))


_T_SPEC = lambda p: pl.BlockSpec((2, p, BM, 128), lambda i: (0, 0, i, 0))
_CNT_SPEC = pl.BlockSpec((2, 2, BM, 128), lambda i: (0, 0, i, 0))
_AB_SPEC = pl.BlockSpec((2, BM, 128), lambda i: (0, i, 0))


def _full_spec(shape):
  nd = len(shape)
  return pl.BlockSpec(shape, lambda i: (0,) * nd)


def _scalars_from_cnt(cnt):
  d = cnt[0, 0] + cnt[1, 0]
  s = cnt[0, 1] + cnt[1, 1]
  deg = 1.0 + d - s
  a = 1.0 / jnp.maximum(deg, 1.0)
  return a, a * (1.0 - s)


def _make_dense(p_in, p_out, form_b, p_next=None, first=False):
  def body(*refs):
    it = iter(refs)
    h_ref = next(it)
    g_ref = next(it) if form_b else None
    t_ref = next(it)
    sc_ref = next(it)
    wok_ref = None if form_b else next(it)
    bop_ref = next(it)
    wrk_ref = next(it)
    wnk_ref = next(it) if p_next is not None else None
    out_ref = next(it)
    gout_ref = next(it) if p_next is not None else None
    ab_ref = next(it) if first else None

    if first:
      a, b = _scalars_from_cnt(sc_ref[...])
      ab_ref[0] = a
      ab_ref[1] = b
    else:
      a = sc_ref[0]
      b = sc_ref[1]

    hs = [h_ref[i] for i in range(p_in)]
    vals = []
    for po in range(p_out):
      if form_b:
        val = a * (t_ref[0, po] + t_ref[1, po]) + b * g_ref[po]
        for pi in range(p_in):
          val = val + _mm(hs[pi], wrk_ref[pi, po])
      else:
        val = None
        for pi in range(p_in):
          agg = a * (t_ref[0, pi] + t_ref[1, pi]) + b * hs[pi]
          term = _mm(agg, wok_ref[pi, po]) + _mm(hs[pi], wrk_ref[pi, po])
          val = term if val is None else val + term
      val = _lrelu(val + bop_ref[po])
      vals.append(val)
      out_ref[po] = val
    if p_next is not None:
      for pn in range(p_next):
        gv = None
        for po in range(p_out):
          term = _mm(vals[po], wnk_ref[po, pn])
          gv = term if gv is None else gv + term
        gout_ref[pn] = gv

  in_specs = [_pk_spec(p_in)]
  if form_b:
    in_specs.append(_pk_spec(p_out))
  in_specs.append(_T_SPEC(p_out if form_b else p_in))
  in_specs.append(_CNT_SPEC if first else _AB_SPEC)
  if not form_b:
    in_specs.append(_full_spec((p_in, p_out, 128, 128)))
  in_specs.append(_full_spec((p_out, 128)))
  in_specs.append(_full_spec((p_in, p_out, 128, 128)))
  if p_next is not None:
    in_specs.append(_full_spec((p_out, p_next, 128, 128)))

  out_shape = [jax.ShapeDtypeStruct((p_out, M, 128), jnp.float32)]
  out_specs = [_pk_spec(p_out)]
  if p_next is not None:
    out_shape.append(jax.ShapeDtypeStruct((p_next, M, 128), jnp.float32))
    out_specs.append(_pk_spec(p_next))
  if first:
    out_shape.append(jax.ShapeDtypeStruct((2, MP, 128), jnp.float32))
    out_specs.append(_AB_SPEC)

  return pl.pallas_call(
      body,
      grid=(GRID,),
      in_specs=in_specs,
      out_specs=out_specs if len(out_specs) > 1 else out_specs[0],
      out_shape=out_shape if len(out_shape) > 1 else out_shape[0],
  )


def _xpack_body(x_ref, p_ref, o_ref):
  o_ref[0] = _mm(x_ref[...], p_ref[...])


_xpack = pl.pallas_call(
    _xpack_body,
    grid=(GRID,),
    in_specs=[pl.BlockSpec((BM, 24), lambda i: (i, 0)),
              _full_spec((24, 128))],
    out_specs=pl.BlockSpec((1, BM, 128), lambda i: (0, i, 0)),
    out_shape=jax.ShapeDtypeStruct((1, M, 128), jnp.float32),
)


def _unpack_body(v_ref, s_ref, o_ref):
  o_ref[...] = _mm(v_ref[0], s_ref[...])


_unpack = pl.pallas_call(
    _unpack_body,
    grid=(GRID,),
    in_specs=[pl.BlockSpec((1, BM, 128), lambda i: (0, i, 0)),
              _full_spec((128, 8))],
    out_specs=pl.BlockSpec((BM, 8), lambda i: (i, 0)),
    out_shape=jax.ShapeDtypeStruct((M, 8), jnp.float32),
)


_scatter1c = _make_scatter_kernel(1, True)
_scatter1 = _make_scatter_kernel(1, False)
_scatter2 = _make_scatter_kernel(2, False)
_scatter4 = _make_scatter_kernel(4, False)

_dense0 = _make_dense(1, 1, False, first=True)
_dense1 = _make_dense(1, 1, False)
_dense2 = _make_dense(1, 2, False)
_dense3 = _make_dense(2, 4, False)
_dense4 = _make_dense(4, 4, False, p_next=2)
_dense5 = _make_dense(4, 2, True, p_next=1)
_dense6 = _make_dense(2, 1, True)
_dense7 = _make_dense(1, 1, False)


def _pad16(w):
  di, do = w.shape
  return jnp.pad(w, ((0, (-di) % 16), (0, (-do) % 16)))


def _kron8(w):
  w = _pad16(w)
  pi, po = w.shape[0] // 16, w.shape[1] // 16
  i8 = jnp.eye(8, dtype=w.dtype)
  return jnp.stack([
      jnp.stack([jnp.kron(i8, w[16 * i:16 * i + 16, 16 * j:16 * j + 16])
                 for j in range(po)])
      for i in range(pi)])


def _packb(bo):
  bo = jnp.pad(bo, (0, (-bo.shape[0]) % 16))
  po = bo.shape[0] // 16
  return jnp.tile(bo.reshape(po, 1, 16), (1, 8, 1)).reshape(po, 128)


def _sc_view(hpk):
  p = hpk.shape[0]
  return hpk.reshape(p, M * 8, 16)


def kernel(x, edge_index,
           Wo0, bo0, Wr0, Wo1, bo1, Wr1, Wo2, bo2, Wr2, Wo3, bo3, Wr3,
           Wo4, bo4, Wr4, Wo5, bo5, Wr5, Wo6, bo6, Wr6, Wo7, bo7, Wr7):
  f32 = jnp.float32
  pad = jnp.stack([jnp.zeros((EP - E,), jnp.int32),
                   jnp.full((EP - E,), N, jnp.int32)])
  ei3 = jnp.concatenate([edge_index, pad], axis=1).reshape(2, EROWS, WIN)
  src = ei3[0]
  dst = ei3[1]
  zeros16 = jnp.zeros((WIN, 16), f32)
  ones16 = jnp.ones((WIN, 16), f32)

  q = jnp.arange(8).repeat(3)
  f = jnp.tile(jnp.arange(3), 8)
  perm = jnp.zeros((24, 128), f32).at[3 * q + f, 16 * q + f].set(1.0)
  xpk = _xpack(x.reshape(M, 24), perm)

  wok = [_kron8(w) for w in (Wo0, Wo1, Wo2, Wo3, Wo4, Wo5, Wo6, Wo7)]
  wrk = [_kron8(w) for w in (Wr0, Wr1, Wr2, Wr3, Wr4, Wr5, Wr6, Wr7)]
  bop = [_packb(b) for b in (bo0, bo1, bo2, bo3, bo4, bo5, bo6, bo7)]

  def t_view(t):
    return t.reshape(t.shape[0], t.shape[1], MP, 128)

  t0, cnt = _scatter1c(_sc_view(xpk), src, dst, zeros16, ones16)
  cntv = cnt.reshape(2, 2, MP, 128)
  h1, ab = _dense0(xpk, t_view(t0), cntv, wok[0], bop[0], wrk[0])
  t1 = _scatter1(_sc_view(h1), src, dst, zeros16, ones16)
  h2 = _dense1(h1, t_view(t1), ab, wok[1], bop[1], wrk[1])
  t2 = _scatter1(_sc_view(h2), src, dst, zeros16, ones16)
  h3 = _dense2(h2, t_view(t2), ab, wok[2], bop[2], wrk[2])
  t3 = _scatter2(_sc_view(h3), src, dst, zeros16, ones16)
  h4 = _dense3(h3, t_view(t3), ab, wok[3], bop[3], wrk[3])
  t4 = _scatter4(_sc_view(h4), src, dst, zeros16, ones16)
  h5, g5 = _dense4(h4, t_view(t4), ab, wok[4], bop[4], wrk[4], wok[5])
  t5 = _scatter2(_sc_view(g5), src, dst, zeros16, ones16)
  h6, g6 = _dense5(h5, g5, t_view(t5), ab, bop[5], wrk[5], wok[6])
  t6 = _scatter1(_sc_view(g6), src, dst, zeros16, ones16)
  h7 = _dense6(h6, g6, t_view(t6), ab, bop[6], wrk[6])
  t7 = _scatter1(_sc_view(h7), src, dst, zeros16, ones16)
  out8 = _dense7(h7, t_view(t7), ab, wok[7], bop[7], wrk[7])
  sel = jnp.zeros((128, 8), f32).at[jnp.arange(8) * 16, jnp.arange(8)].set(1.0)
  return _unpack(out8, sel).reshape(N)


# speedup vs baseline: 20.4421x; 1.2241x over previous
import functools

import jax
import jax.numpy as jnp
from jax import lax
from jax.experimental import pallas as pl
from jax.experimental.pallas import tpu as pltpu
from jax.experimental.pallas import tpu_sc as plsc

N = 100000
E = 1600000
WIN = 128
NB = 4
WPT = 392
NGRP = WPT // NB
NTILES = 32
EP = NTILES * WPT * WIN
EROWS = EP // WIN
ROWS_PT = 6272
NP = 16 * ROWS_PT
M = N // 8
MP = NP // 8
BM = 256
GRID = (M + BM - 1) // BM


def _make_scatter_kernel(p, with_counts):
  mesh = plsc.VectorSubcoreMesh(core_axis_name="c", subcore_axis_name="s",
                                num_cores=2, num_subcores=16)
  t_type = jax.ShapeDtypeStruct((2, p, NP, 16), jnp.float32)
  if with_counts:
    out_type = [t_type, jax.ShapeDtypeStruct((2, 2, NP, 16), jnp.float32)]
  else:
    out_type = t_type

  @functools.partial(
      pl.kernel,
      out_type=out_type,
      mesh=mesh,
      compiler_params=pltpu.CompilerParams(use_tc_tiling_on_sc=False),
      scratch_types=[
          pltpu.VMEM((2 * NB, WIN), jnp.int32),
          pltpu.VMEM((2 * NB, WIN), jnp.int32),
          pltpu.VMEM((2 * NB, WIN), jnp.int32),
          pltpu.VMEM((2 * NB, WIN, 16), jnp.float32),
          pltpu.VMEM((WIN, 16), jnp.float32),
          pltpu.VMEM((WIN, 16), jnp.float32),
          pltpu.VMEM_SHARED((NP, 16), jnp.float32),
          pltpu.SemaphoreType.DMA((NB,)),
          pltpu.SemaphoreType.DMA((2 * NB,)),
      ],
  )
  def scatter_kernel(table, src2, dst2, zeros_hbm, ones_hbm, *rest):
    if with_counts:
      out, cnt_out = rest[0], rest[1]
      scratch = rest[2:]
    else:
      out = rest[0]
      cnt_out = None
      scratch = rest[1:]
    sbuf, dbuf, selbuf, rows, zeros_v, ones_v, accum, sems, ssems = scratch
    c = lax.axis_index("c")
    s = lax.axis_index("s")
    erow0 = (c * 16 + s) * WPT
    r0 = s * ROWS_PT
    pltpu.sync_copy(zeros_hbm, zeros_v)
    if with_counts:
      pltpu.sync_copy(ones_hbm, ones_v)

    def zero_accum():
      for k in range(ROWS_PT // WIN):
        pltpu.sync_copy(zeros_v, accum.at[pl.ds(r0 + k * WIN, WIN)])

    def flush(dst_ref):
      sl = pl.ds(r0, ROWS_PT)
      pltpu.sync_copy(accum.at[sl], dst_ref.at[c, sl])

    def load_group(g, par, want_src):
      base = erow0 + g * NB
      if want_src:
        pltpu.sync_copy(src2.at[pl.ds(base, NB)],
                        sbuf.at[pl.ds(par * NB, NB)])
      pltpu.sync_copy(dst2.at[pl.ds(base, NB)],
                      dbuf.at[pl.ds(par * NB, NB)])

    def drain_scatters(par):
      for b in range(NB):
        i = par * NB + b
        pltpu.make_async_copy(rows.at[i], accum.at[dbuf.at[i]],
                              ssems.at[i]).wait()

    for j in range(p):
      tbl = table.at[j]
      zero_accum()
      plsc.subcore_barrier()

      load_group(0, 0, True)
      for b in range(NB):
        pltpu.async_copy(tbl.at[sbuf.at[b]], rows.at[b], sems.at[b])

      def one_group(g, par):
        nxt = 1 - par

        @pl.when(g > 0)
        def _():
          drain_scatters(nxt)

        @pl.when(g + 1 < NGRP)
        def _():
          load_group(g + 1, nxt, True)

        for b in range(NB):
          i = par * NB + b
          pltpu.make_async_copy(tbl.at[sbuf.at[i]], rows.at[i],
                                sems.at[b]).wait()
          pltpu.async_copy(rows.at[i], accum.at[dbuf.at[i]], ssems.at[i],
                           add=True)

          @pl.when(g + 1 < NGRP)
          def _():
            ni = nxt * NB + b
            pltpu.async_copy(tbl.at[sbuf.at[ni]], rows.at[ni], sems.at[b])

      def body(i, carry):
        one_group(2 * i, 0)
        one_group(2 * i + 1, 1)
        return carry

      lax.fori_loop(0, NGRP // 2, body, 0)
      drain_scatters((NGRP - 1) % 2)
      plsc.subcore_barrier()
      flush(out.at[slice(None), j])

    if with_counts:
      def drain_ones(par, idx_buf):
        for b in range(NB):
          i = par * NB + b
          pltpu.make_async_copy(ones_v, accum.at[idx_buf.at[i]],
                                ssems.at[i]).wait()

      zero_accum()
      plsc.subcore_barrier()
      load_group(0, 0, False)

      def deg_group(g, par):
        nxt = 1 - par

        @pl.when(g > 0)
        def _():
          drain_ones(nxt, dbuf)

        @pl.when(g + 1 < NGRP)
        def _():
          load_group(g + 1, nxt, False)

        for b in range(NB):
          i = par * NB + b
          pltpu.async_copy(ones_v, accum.at[dbuf.at[i]], ssems.at[i],
                           add=True)

      def deg_body(i, carry):
        deg_group(2 * i, 0)
        deg_group(2 * i + 1, 1)
        return carry

      lax.fori_loop(0, NGRP // 2, deg_body, 0)
      drain_ones((NGRP - 1) % 2, dbuf)
      plsc.subcore_barrier()
      flush(cnt_out.at[slice(None), 0])

      zero_accum()
      plsc.subcore_barrier()
      iota16 = lax.iota(jnp.int32, 16)
      trash = [N + (k * 16) % 256 + iota16 for k in range(WIN // 16)]
      load_group(0, 0, True)

      def self_group(g, par):
        nxt = 1 - par

        @pl.when(g > 0)
        def _():
          drain_ones(nxt, selbuf)

        @pl.when(g + 1 < NGRP)
        def _():
          load_group(g + 1, nxt, True)

        for b in range(NB):
          i = par * NB + b
          for k in range(WIN // 16):
            sv = sbuf[i, pl.ds(k * 16, 16)]
            dv = dbuf[i, pl.ds(k * 16, 16)]
            selbuf[i, pl.ds(k * 16, 16)] = jnp.where(sv == dv, dv, trash[k])
          pltpu.async_copy(ones_v, accum.at[selbuf.at[i]], ssems.at[i],
                           add=True)

      def self_body(i, carry):
        self_group(2 * i, 0)
        self_group(2 * i + 1, 1)
        return carry

      lax.fori_loop(0, NGRP // 2, self_body, 0)
      drain_ones((NGRP - 1) % 2, selbuf)
      plsc.subcore_barrier()
      flush(cnt_out.at[slice(None), 1])

  return scatter_kernel


def _lrelu(v):
  return jnp.where(v >= 0.0, v, 0.1 * v)


def _mm(a, w):
  return jax.lax.dot_general(a, w, (((1,), (0,)), ((), ())),
                             precision=jax.lax.Precision.HIGHEST,
                             preferred_element_type=jnp.float32)


def _pk_spec(p):
  return pl.BlockSpec((p, BM, 128), lambda i: (0, i, 0))


_T_SPEC = lambda p: pl.BlockSpec((2, p, BM, 128), lambda i: (0, 0, i, 0))
_CNT_SPEC = pl.BlockSpec((2, 2, BM, 128), lambda i: (0, 0, i, 0))
_AB_SPEC = pl.BlockSpec((2, BM, 128), lambda i: (0, i, 0))


def _full_spec(shape):
  nd = len(shape)
  return pl.BlockSpec(shape, lambda i: (0,) * nd)


def _scalars_from_cnt(cnt):
  d = cnt[0, 0] + cnt[1, 0]
  s = cnt[0, 1] + cnt[1, 1]
  deg = 1.0 + d - s
  a = 1.0 / jnp.maximum(deg, 1.0)
  return a, a * (1.0 - s)


def _make_dense(p_in, p_out, form_b, p_next=None, first=False):
  def body(*refs):
    it = iter(refs)
    h_ref = next(it)
    g_ref = next(it) if form_b else None
    t_ref = next(it)
    sc_ref = next(it)
    wok_ref = None if form_b else next(it)
    bop_ref = next(it)
    wrk_ref = next(it)
    wnk_ref = next(it) if p_next is not None else None
    out_ref = next(it)
    gout_ref = next(it) if p_next is not None else None
    ab_ref = next(it) if first else None

    if first:
      a, b = _scalars_from_cnt(sc_ref[...])
      ab_ref[0] = a
      ab_ref[1] = b
    else:
      a = sc_ref[0]
      b = sc_ref[1]

    hs = [h_ref[i] for i in range(p_in)]
    vals = []
    for po in range(p_out):
      if form_b:
        val = a * (t_ref[0, po] + t_ref[1, po]) + b * g_ref[po]
        for pi in range(p_in):
          val = val + _mm(hs[pi], wrk_ref[pi, po])
      else:
        val = None
        for pi in range(p_in):
          agg = a * (t_ref[0, pi] + t_ref[1, pi]) + b * hs[pi]
          term = _mm(agg, wok_ref[pi, po]) + _mm(hs[pi], wrk_ref[pi, po])
          val = term if val is None else val + term
      val = _lrelu(val + bop_ref[po])
      vals.append(val)
      out_ref[po] = val
    if p_next is not None:
      for pn in range(p_next):
        gv = None
        for po in range(p_out):
          term = _mm(vals[po], wnk_ref[po, pn])
          gv = term if gv is None else gv + term
        gout_ref[pn] = gv

  in_specs = [_pk_spec(p_in)]
  if form_b:
    in_specs.append(_pk_spec(p_out))
  in_specs.append(_T_SPEC(p_out if form_b else p_in))
  in_specs.append(_CNT_SPEC if first else _AB_SPEC)
  if not form_b:
    in_specs.append(_full_spec((p_in, p_out, 128, 128)))
  in_specs.append(_full_spec((p_out, 128)))
  in_specs.append(_full_spec((p_in, p_out, 128, 128)))
  if p_next is not None:
    in_specs.append(_full_spec((p_out, p_next, 128, 128)))

  out_shape = [jax.ShapeDtypeStruct((p_out, M, 128), jnp.float32)]
  out_specs = [_pk_spec(p_out)]
  if p_next is not None:
    out_shape.append(jax.ShapeDtypeStruct((p_next, M, 128), jnp.float32))
    out_specs.append(_pk_spec(p_next))
  if first:
    out_shape.append(jax.ShapeDtypeStruct((2, MP, 128), jnp.float32))
    out_specs.append(_AB_SPEC)

  return pl.pallas_call(
      body,
      grid=(GRID,),
      in_specs=in_specs,
      out_specs=out_specs if len(out_specs) > 1 else out_specs[0],
      out_shape=out_shape if len(out_shape) > 1 else out_shape[0],
  )


def _xpack_body(x_ref, p_ref, o_ref):
  o_ref[0] = _mm(x_ref[...], p_ref[...])


_xpack = pl.pallas_call(
    _xpack_body,
    grid=(GRID,),
    in_specs=[pl.BlockSpec((BM, 24), lambda i: (i, 0)),
              _full_spec((24, 128))],
    out_specs=pl.BlockSpec((1, BM, 128), lambda i: (0, i, 0)),
    out_shape=jax.ShapeDtypeStruct((1, M, 128), jnp.float32),
)


def _unpack_body(v_ref, s_ref, o_ref):
  o_ref[...] = _mm(v_ref[0], s_ref[...])


_unpack = pl.pallas_call(
    _unpack_body,
    grid=(GRID,),
    in_specs=[pl.BlockSpec((1, BM, 128), lambda i: (0, i, 0)),
              _full_spec((128, 8))],
    out_specs=pl.BlockSpec((BM, 8), lambda i: (i, 0)),
    out_shape=jax.ShapeDtypeStruct((M, 8), jnp.float32),
)


_scatter1c = _make_scatter_kernel(1, True)
_scatter1 = _make_scatter_kernel(1, False)
_scatter2 = _make_scatter_kernel(2, False)
_scatter4 = _make_scatter_kernel(4, False)

_dense0 = _make_dense(1, 1, False, first=True)
_dense1 = _make_dense(1, 1, False)
_dense2 = _make_dense(1, 2, False)
_dense3 = _make_dense(2, 4, False)
_dense4 = _make_dense(4, 4, False, p_next=2)
_dense5 = _make_dense(4, 2, True, p_next=1)
_dense6 = _make_dense(2, 1, True)
_dense7 = _make_dense(1, 1, False)


def _pad16(w):
  di, do = w.shape
  return jnp.pad(w, ((0, (-di) % 16), (0, (-do) % 16)))


def _kron8(w):
  w = _pad16(w)
  pi, po = w.shape[0] // 16, w.shape[1] // 16
  i8 = jnp.eye(8, dtype=w.dtype)
  return jnp.stack([
      jnp.stack([jnp.kron(i8, w[16 * i:16 * i + 16, 16 * j:16 * j + 16])
                 for j in range(po)])
      for i in range(pi)])


def _packb(bo):
  bo = jnp.pad(bo, (0, (-bo.shape[0]) % 16))
  po = bo.shape[0] // 16
  return jnp.tile(bo.reshape(po, 1, 16), (1, 8, 1)).reshape(po, 128)


def _sc_view(hpk):
  p = hpk.shape[0]
  return hpk.reshape(p, M * 8, 16)


def kernel(x, edge_index,
           Wo0, bo0, Wr0, Wo1, bo1, Wr1, Wo2, bo2, Wr2, Wo3, bo3, Wr3,
           Wo4, bo4, Wr4, Wo5, bo5, Wr5, Wo6, bo6, Wr6, Wo7, bo7, Wr7):
  f32 = jnp.float32
  pad = jnp.stack([jnp.zeros((EP - E,), jnp.int32),
                   jnp.full((EP - E,), N, jnp.int32)])
  ei3 = jnp.concatenate([edge_index, pad], axis=1).reshape(2, EROWS, WIN)
  src = ei3[0]
  dst = ei3[1]
  zeros16 = jnp.zeros((WIN, 16), f32)
  ones16 = jnp.ones((WIN, 16), f32)

  q = jnp.arange(8).repeat(3)
  f = jnp.tile(jnp.arange(3), 8)
  perm = jnp.zeros((24, 128), f32).at[3 * q + f, 16 * q + f].set(1.0)
  xpk = _xpack(x.reshape(M, 24), perm)

  wok = [_kron8(w) for w in (Wo0, Wo1, Wo2, Wo3, Wo4, Wo5, Wo6, Wo7)]
  wrk = [_kron8(w) for w in (Wr0, Wr1, Wr2, Wr3, Wr4, Wr5, Wr6, Wr7)]
  bop = [_packb(b) for b in (bo0, bo1, bo2, bo3, bo4, bo5, bo6, bo7)]

  def t_view(t):
    return t.reshape(t.shape[0], t.shape[1], MP, 128)

  t0, cnt = _scatter1c(_sc_view(xpk), src, dst, zeros16, ones16)
  cntv = cnt.reshape(2, 2, MP, 128)
  h1, ab = _dense0(xpk, t_view(t0), cntv, wok[0], bop[0], wrk[0])
  t1 = _scatter1(_sc_view(h1), src, dst, zeros16, ones16)
  h2 = _dense1(h1, t_view(t1), ab, wok[1], bop[1], wrk[1])
  t2 = _scatter1(_sc_view(h2), src, dst, zeros16, ones16)
  h3 = _dense2(h2, t_view(t2), ab, wok[2], bop[2], wrk[2])
  t3 = _scatter2(_sc_view(h3), src, dst, zeros16, ones16)
  h4 = _dense3(h3, t_view(t3), ab, wok[3], bop[3], wrk[3])
  t4 = _scatter4(_sc_view(h4), src, dst, zeros16, ones16)
  h5, g5 = _dense4(h4, t_view(t4), ab, wok[4], bop[4], wrk[4], wok[5])
  t5 = _scatter2(_sc_view(g5), src, dst, zeros16, ones16)
  h6, g6 = _dense5(h5, g5, t_view(t5), ab, bop[5], wrk[5], wok[6])
  t6 = _scatter1(_sc_view(g6), src, dst, zeros16, ones16)
  h7 = _dense6(h6, g6, t_view(t6), ab, bop[6], wrk[6])
  t7 = _scatter1(_sc_view(h7), src, dst, zeros16, ones16)
  out8 = _dense7(h7, t_view(t7), ab, wok[7], bop[7], wrk[7])
  sel = jnp.zeros((128, 8), f32).at[jnp.arange(8) * 16, jnp.arange(8)].set(1.0)
  return _unpack(out8, sel).reshape(N)


# speedup vs baseline: 23.0548x; 1.1278x over previous
import functools

import jax
import jax.numpy as jnp
from jax import lax
from jax.experimental import pallas as pl
from jax.experimental.pallas import tpu as pltpu
from jax.experimental.pallas import tpu_sc as plsc

N = 100000
E = 1600000
WIN = 128
NB = 4
WPT = 392
NGRP = WPT // NB
NTILES = 32
EP = NTILES * WPT * WIN
EROWS = EP // WIN
ROWS_PT = 6272
NP = 16 * ROWS_PT
M = N // 8
MP = NP // 8
BM = 256
GRID = (M + BM - 1) // BM


def _make_scatter_kernel(p, with_counts):
  mesh = plsc.VectorSubcoreMesh(core_axis_name="c", subcore_axis_name="s",
                                num_cores=2, num_subcores=16)
  t_type = jax.ShapeDtypeStruct((2, p, NP, 16), jnp.float32)
  if with_counts:
    out_type = [t_type, jax.ShapeDtypeStruct((2, 2, NP, 16), jnp.float32)]
  else:
    out_type = t_type

  @functools.partial(
      pl.kernel,
      out_type=out_type,
      mesh=mesh,
      compiler_params=pltpu.CompilerParams(use_tc_tiling_on_sc=False),
      scratch_types=[
          pltpu.VMEM((2 * NB, WIN), jnp.int32),
          pltpu.VMEM((2 * NB, WIN), jnp.int32),
          pltpu.VMEM((2 * NB, WIN), jnp.int32),
          pltpu.VMEM((2 * NB, WIN, 16), jnp.float32),
          pltpu.VMEM((WIN, 16), jnp.float32),
          pltpu.VMEM((WIN, 16), jnp.float32),
          pltpu.VMEM_SHARED((NP, 16), jnp.float32),
          pltpu.SemaphoreType.DMA((NB,)),
          pltpu.SemaphoreType.DMA((2 * NB,)),
          pltpu.SemaphoreType.DMA,
      ],
  )
  def scatter_kernel(table, src2, dst2, zeros_hbm, ones_hbm, *rest):
    if with_counts:
      out, cnt_out = rest[0], rest[1]
      scratch = rest[2:]
    else:
      out = rest[0]
      cnt_out = None
      scratch = rest[1:]
    (sbuf, dbuf, selbuf, rows, zeros_v, ones_v, accum, sems, ssems,
     isem) = scratch
    c = lax.axis_index("c")
    s = lax.axis_index("s")
    erow0 = (c * 16 + s) * WPT
    r0 = s * ROWS_PT
    pltpu.sync_copy(zeros_hbm, zeros_v)
    if with_counts:
      pltpu.sync_copy(ones_hbm, ones_v)

    def zero_accum():
      for k in range(ROWS_PT // WIN):
        pltpu.sync_copy(zeros_v, accum.at[pl.ds(r0 + k * WIN, WIN)])

    def flush(dst_ref):
      sl = pl.ds(r0, ROWS_PT)
      pltpu.sync_copy(accum.at[sl], dst_ref.at[c, sl])

    def load_group(g, par, want_src):
      base = erow0 + g * NB
      if want_src:
        pltpu.sync_copy(src2.at[pl.ds(base, NB)],
                        sbuf.at[pl.ds(par * NB, NB)])
      pltpu.sync_copy(dst2.at[pl.ds(base, NB)],
                      dbuf.at[pl.ds(par * NB, NB)])

    def drain_scatters(par):
      for b in range(NB):
        i = par * NB + b
        pltpu.make_async_copy(rows.at[i], accum.at[dbuf.at[i]],
                              ssems.at[i]).wait()

    for j in range(p):
      tbl = table.at[j]
      zero_accum()
      plsc.subcore_barrier()

      load_group(0, 0, True)
      for b in range(NB):
        pltpu.async_copy(tbl.at[sbuf.at[b]], rows.at[b], sems.at[b])

      def one_group(g, par):
        nxt = 1 - par
        base = erow0 + (g + 1) * NB
        nsl = pl.ds(nxt * NB, NB)

        @pl.when(g > 0)
        def _():
          drain_scatters(nxt)

        @pl.when(g + 1 < NGRP)
        def _():
          pltpu.async_copy(src2.at[pl.ds(base, NB)], sbuf.at[nsl], isem)
          pltpu.async_copy(dst2.at[pl.ds(base, NB)], dbuf.at[nsl], isem)

        for b in range(NB):
          i = par * NB + b
          pltpu.make_async_copy(tbl.at[sbuf.at[i]], rows.at[i],
                                sems.at[b]).wait()
          pltpu.async_copy(rows.at[i], accum.at[dbuf.at[i]], ssems.at[i],
                           add=True)

        @pl.when(g + 1 < NGRP)
        def _():
          pltpu.make_async_copy(src2.at[pl.ds(base, NB)], sbuf.at[nsl],
                                isem).wait()
          pltpu.make_async_copy(dst2.at[pl.ds(base, NB)], dbuf.at[nsl],
                                isem).wait()
          for b in range(NB):
            ni = nxt * NB + b
            pltpu.async_copy(tbl.at[sbuf.at[ni]], rows.at[ni], sems.at[b])

      def body(i, carry):
        one_group(2 * i, 0)
        one_group(2 * i + 1, 1)
        return carry

      lax.fori_loop(0, NGRP // 2, body, 0)
      drain_scatters((NGRP - 1) % 2)
      plsc.subcore_barrier()
      flush(out.at[slice(None), j])

    if with_counts:
      def drain_ones(par, idx_buf):
        for b in range(NB):
          i = par * NB + b
          pltpu.make_async_copy(ones_v, accum.at[idx_buf.at[i]],
                                ssems.at[i]).wait()

      zero_accum()
      plsc.subcore_barrier()
      load_group(0, 0, False)

      def deg_group(g, par):
        nxt = 1 - par

        @pl.when(g > 0)
        def _():
          drain_ones(nxt, dbuf)

        @pl.when(g + 1 < NGRP)
        def _():
          load_group(g + 1, nxt, False)

        for b in range(NB):
          i = par * NB + b
          pltpu.async_copy(ones_v, accum.at[dbuf.at[i]], ssems.at[i],
                           add=True)

      def deg_body(i, carry):
        deg_group(2 * i, 0)
        deg_group(2 * i + 1, 1)
        return carry

      lax.fori_loop(0, NGRP // 2, deg_body, 0)
      drain_ones((NGRP - 1) % 2, dbuf)
      plsc.subcore_barrier()
      flush(cnt_out.at[slice(None), 0])

      zero_accum()
      plsc.subcore_barrier()
      iota16 = lax.iota(jnp.int32, 16)
      trash = [N + (k * 16) % 256 + iota16 for k in range(WIN // 16)]
      load_group(0, 0, True)

      def self_group(g, par):
        nxt = 1 - par

        @pl.when(g > 0)
        def _():
          drain_ones(nxt, selbuf)

        @pl.when(g + 1 < NGRP)
        def _():
          load_group(g + 1, nxt, True)

        for b in range(NB):
          i = par * NB + b
          for k in range(WIN // 16):
            sv = sbuf[i, pl.ds(k * 16, 16)]
            dv = dbuf[i, pl.ds(k * 16, 16)]
            selbuf[i, pl.ds(k * 16, 16)] = jnp.where(sv == dv, dv, trash[k])
          pltpu.async_copy(ones_v, accum.at[selbuf.at[i]], ssems.at[i],
                           add=True)

      def self_body(i, carry):
        self_group(2 * i, 0)
        self_group(2 * i + 1, 1)
        return carry

      lax.fori_loop(0, NGRP // 2, self_body, 0)
      drain_ones((NGRP - 1) % 2, selbuf)
      plsc.subcore_barrier()
      flush(cnt_out.at[slice(None), 1])

  return scatter_kernel


def _lrelu(v):
  return jnp.where(v >= 0.0, v, 0.1 * v)


def _mm(a, w):
  return jax.lax.dot_general(a, w, (((1,), (0,)), ((), ())),
                             precision=jax.lax.Precision.HIGHEST,
                             preferred_element_type=jnp.float32)


def _pk_spec(p):
  return pl.BlockSpec((p, BM, 128), lambda i: (0, i, 0))


_T_SPEC = lambda p: pl.BlockSpec((2, p, BM, 128), lambda i: (0, 0, i, 0))
_CNT_SPEC = pl.BlockSpec((2, 2, BM, 128), lambda i: (0, 0, i, 0))
_AB_SPEC = pl.BlockSpec((2, BM, 128), lambda i: (0, i, 0))


def _full_spec(shape):
  nd = len(shape)
  return pl.BlockSpec(shape, lambda i: (0,) * nd)


def _scalars_from_cnt(cnt):
  d = cnt[0, 0] + cnt[1, 0]
  s = cnt[0, 1] + cnt[1, 1]
  deg = 1.0 + d - s
  a = 1.0 / jnp.maximum(deg, 1.0)
  return a, a * (1.0 - s)


def _make_dense(p_in, p_out, form_b, p_next=None, first=False):
  def body(*refs):
    it = iter(refs)
    h_ref = next(it)
    g_ref = next(it) if form_b else None
    t_ref = next(it)
    sc_ref = next(it)
    wok_ref = None if form_b else next(it)
    bop_ref = next(it)
    wrk_ref = next(it)
    wnk_ref = next(it) if p_next is not None else None
    out_ref = next(it)
    gout_ref = next(it) if p_next is not None else None
    ab_ref = next(it) if first else None

    if first:
      a, b = _scalars_from_cnt(sc_ref[...])
      ab_ref[0] = a
      ab_ref[1] = b
    else:
      a = sc_ref[0]
      b = sc_ref[1]

    hs = [h_ref[i] for i in range(p_in)]
    vals = []
    for po in range(p_out):
      if form_b:
        val = a * (t_ref[0, po] + t_ref[1, po]) + b * g_ref[po]
        for pi in range(p_in):
          val = val + _mm(hs[pi], wrk_ref[pi, po])
      else:
        val = None
        for pi in range(p_in):
          agg = a * (t_ref[0, pi] + t_ref[1, pi]) + b * hs[pi]
          term = _mm(agg, wok_ref[pi, po]) + _mm(hs[pi], wrk_ref[pi, po])
          val = term if val is None else val + term
      val = _lrelu(val + bop_ref[po])
      vals.append(val)
      out_ref[po] = val
    if p_next is not None:
      for pn in range(p_next):
        gv = None
        for po in range(p_out):
          term = _mm(vals[po], wnk_ref[po, pn])
          gv = term if gv is None else gv + term
        gout_ref[pn] = gv

  in_specs = [_pk_spec(p_in)]
  if form_b:
    in_specs.append(_pk_spec(p_out))
  in_specs.append(_T_SPEC(p_out if form_b else p_in))
  in_specs.append(_CNT_SPEC if first else _AB_SPEC)
  if not form_b:
    in_specs.append(_full_spec((p_in, p_out, 128, 128)))
  in_specs.append(_full_spec((p_out, 128)))
  in_specs.append(_full_spec((p_in, p_out, 128, 128)))
  if p_next is not None:
    in_specs.append(_full_spec((p_out, p_next, 128, 128)))

  out_shape = [jax.ShapeDtypeStruct((p_out, M, 128), jnp.float32)]
  out_specs = [_pk_spec(p_out)]
  if p_next is not None:
    out_shape.append(jax.ShapeDtypeStruct((p_next, M, 128), jnp.float32))
    out_specs.append(_pk_spec(p_next))
  if first:
    out_shape.append(jax.ShapeDtypeStruct((2, MP, 128), jnp.float32))
    out_specs.append(_AB_SPEC)

  return pl.pallas_call(
      body,
      grid=(GRID,),
      in_specs=in_specs,
      out_specs=out_specs if len(out_specs) > 1 else out_specs[0],
      out_shape=out_shape if len(out_shape) > 1 else out_shape[0],
  )


def _xpack_body(x_ref, p_ref, o_ref):
  o_ref[0] = _mm(x_ref[...], p_ref[...])


_xpack = pl.pallas_call(
    _xpack_body,
    grid=(GRID,),
    in_specs=[pl.BlockSpec((BM, 24), lambda i: (i, 0)),
              _full_spec((24, 128))],
    out_specs=pl.BlockSpec((1, BM, 128), lambda i: (0, i, 0)),
    out_shape=jax.ShapeDtypeStruct((1, M, 128), jnp.float32),
)


def _unpack_body(v_ref, s_ref, o_ref):
  o_ref[...] = _mm(v_ref[0], s_ref[...])


_unpack = pl.pallas_call(
    _unpack_body,
    grid=(GRID,),
    in_specs=[pl.BlockSpec((1, BM, 128), lambda i: (0, i, 0)),
              _full_spec((128, 8))],
    out_specs=pl.BlockSpec((BM, 8), lambda i: (i, 0)),
    out_shape=jax.ShapeDtypeStruct((M, 8), jnp.float32),
)


_scatter1c = _make_scatter_kernel(1, True)
_scatter1 = _make_scatter_kernel(1, False)
_scatter2 = _make_scatter_kernel(2, False)
_scatter4 = _make_scatter_kernel(4, False)

_dense0 = _make_dense(1, 1, False, first=True)
_dense1 = _make_dense(1, 1, False)
_dense2 = _make_dense(1, 2, False)
_dense3 = _make_dense(2, 4, False)
_dense4 = _make_dense(4, 4, False, p_next=2)
_dense5 = _make_dense(4, 2, True, p_next=1)
_dense6 = _make_dense(2, 1, True)
_dense7 = _make_dense(1, 1, False)


def _pad16(w):
  di, do = w.shape
  return jnp.pad(w, ((0, (-di) % 16), (0, (-do) % 16)))


def _kron8(w):
  w = _pad16(w)
  pi, po = w.shape[0] // 16, w.shape[1] // 16
  i8 = jnp.eye(8, dtype=w.dtype)
  return jnp.stack([
      jnp.stack([jnp.kron(i8, w[16 * i:16 * i + 16, 16 * j:16 * j + 16])
                 for j in range(po)])
      for i in range(pi)])


def _packb(bo):
  bo = jnp.pad(bo, (0, (-bo.shape[0]) % 16))
  po = bo.shape[0] // 16
  return jnp.tile(bo.reshape(po, 1, 16), (1, 8, 1)).reshape(po, 128)


def _sc_view(hpk):
  p = hpk.shape[0]
  return hpk.reshape(p, M * 8, 16)


def kernel(x, edge_index,
           Wo0, bo0, Wr0, Wo1, bo1, Wr1, Wo2, bo2, Wr2, Wo3, bo3, Wr3,
           Wo4, bo4, Wr4, Wo5, bo5, Wr5, Wo6, bo6, Wr6, Wo7, bo7, Wr7):
  f32 = jnp.float32
  pad = jnp.stack([jnp.zeros((EP - E,), jnp.int32),
                   jnp.full((EP - E,), N, jnp.int32)])
  ei3 = jnp.concatenate([edge_index, pad], axis=1).reshape(2, EROWS, WIN)
  src = ei3[0]
  dst = ei3[1]
  zeros16 = jnp.zeros((WIN, 16), f32)
  ones16 = jnp.ones((WIN, 16), f32)

  q = jnp.arange(8).repeat(3)
  f = jnp.tile(jnp.arange(3), 8)
  perm = jnp.zeros((24, 128), f32).at[3 * q + f, 16 * q + f].set(1.0)
  xpk = _xpack(x.reshape(M, 24), perm)

  wok = [_kron8(w) for w in (Wo0, Wo1, Wo2, Wo3, Wo4, Wo5, Wo6, Wo7)]
  wrk = [_kron8(w) for w in (Wr0, Wr1, Wr2, Wr3, Wr4, Wr5, Wr6, Wr7)]
  bop = [_packb(b) for b in (bo0, bo1, bo2, bo3, bo4, bo5, bo6, bo7)]

  def t_view(t):
    return t.reshape(t.shape[0], t.shape[1], MP, 128)

  t0, cnt = _scatter1c(_sc_view(xpk), src, dst, zeros16, ones16)
  cntv = cnt.reshape(2, 2, MP, 128)
  h1, ab = _dense0(xpk, t_view(t0), cntv, wok[0], bop[0], wrk[0])
  t1 = _scatter1(_sc_view(h1), src, dst, zeros16, ones16)
  h2 = _dense1(h1, t_view(t1), ab, wok[1], bop[1], wrk[1])
  t2 = _scatter1(_sc_view(h2), src, dst, zeros16, ones16)
  h3 = _dense2(h2, t_view(t2), ab, wok[2], bop[2], wrk[2])
  t3 = _scatter2(_sc_view(h3), src, dst, zeros16, ones16)
  h4 = _dense3(h3, t_view(t3), ab, wok[3], bop[3], wrk[3])
  t4 = _scatter4(_sc_view(h4), src, dst, zeros16, ones16)
  h5, g5 = _dense4(h4, t_view(t4), ab, wok[4], bop[4], wrk[4], wok[5])
  t5 = _scatter2(_sc_view(g5), src, dst, zeros16, ones16)
  h6, g6 = _dense5(h5, g5, t_view(t5), ab, bop[5], wrk[5], wok[6])
  t6 = _scatter1(_sc_view(g6), src, dst, zeros16, ones16)
  h7 = _dense6(h6, g6, t_view(t6), ab, bop[6], wrk[6])
  t7 = _scatter1(_sc_view(h7), src, dst, zeros16, ones16)
  out8 = _dense7(h7, t_view(t7), ab, wok[7], bop[7], wrk[7])
  sel = jnp.zeros((128, 8), f32).at[jnp.arange(8) * 16, jnp.arange(8)].set(1.0)
  return _unpack(out8, sel).reshape(N)


# speedup vs baseline: 23.9630x; 1.0394x over previous
import functools

import jax
import jax.numpy as jnp
from jax import lax
from jax.experimental import pallas as pl
from jax.experimental.pallas import tpu as pltpu
from jax.experimental.pallas import tpu_sc as plsc

N = 100000
E = 1600000
WIN = 128
NB = 4
WPT = 392
NGRP = WPT // NB
NTILES = 32
EP = NTILES * WPT * WIN
EROWS = EP // WIN
ROWS_PT = 6272
NP = 16 * ROWS_PT
M = N // 8
MP = NP // 8
BM = 256
GRID = (M + BM - 1) // BM


def _make_scatter_kernel(p, with_counts):
  mesh = plsc.VectorSubcoreMesh(core_axis_name="c", subcore_axis_name="s",
                                num_cores=2, num_subcores=16)
  t_type = jax.ShapeDtypeStruct((2, p, NP, 16), jnp.float32)
  if with_counts:
    out_type = [t_type, jax.ShapeDtypeStruct((2, 2, NP, 16), jnp.float32)]
  else:
    out_type = t_type

  @functools.partial(
      pl.kernel,
      out_type=out_type,
      mesh=mesh,
      compiler_params=pltpu.CompilerParams(use_tc_tiling_on_sc=False),
      scratch_types=[
          pltpu.VMEM((2 * NB, WIN), jnp.int32),
          pltpu.VMEM((2 * NB, WIN), jnp.int32),
          pltpu.VMEM((2 * NB, WIN), jnp.int32),
          pltpu.VMEM((2 * NB, WIN, 16), jnp.float32),
          pltpu.VMEM((WIN, 16), jnp.float32),
          pltpu.VMEM((WIN, 16), jnp.float32),
          pltpu.VMEM_SHARED((NP, 16), jnp.float32),
          pltpu.SemaphoreType.DMA((NB,)),
          pltpu.SemaphoreType.DMA((2 * NB,)),
          pltpu.SemaphoreType.DMA,
      ],
  )
  def scatter_kernel(table, src2, dst2, zeros_hbm, ones_hbm, *rest):
    if with_counts:
      out, cnt_out = rest[0], rest[1]
      scratch = rest[2:]
    else:
      out = rest[0]
      cnt_out = None
      scratch = rest[1:]
    (sbuf, dbuf, selbuf, rows, zeros_v, ones_v, accum, sems, ssems,
     isem) = scratch
    c = lax.axis_index("c")
    s = lax.axis_index("s")
    erow0 = (c * 16 + s) * WPT
    r0 = s * ROWS_PT
    pltpu.sync_copy(zeros_hbm, zeros_v)
    if with_counts:
      pltpu.sync_copy(ones_hbm, ones_v)

    def zero_accum():
      for k in range(ROWS_PT // WIN):
        pltpu.sync_copy(zeros_v, accum.at[pl.ds(r0 + k * WIN, WIN)])

    def flush(dst_ref):
      sl = pl.ds(r0, ROWS_PT)
      pltpu.sync_copy(accum.at[sl], dst_ref.at[c, sl])

    def load_group(g, par, want_src):
      base = erow0 + g * NB
      if want_src:
        pltpu.sync_copy(src2.at[pl.ds(base, NB)],
                        sbuf.at[pl.ds(par * NB, NB)])
      pltpu.sync_copy(dst2.at[pl.ds(base, NB)],
                      dbuf.at[pl.ds(par * NB, NB)])

    def drain_scatters(par):
      for b in range(NB):
        i = par * NB + b
        pltpu.make_async_copy(rows.at[i], accum.at[dbuf.at[i]],
                              ssems.at[i]).wait()

    for j in range(p):
      tbl = table.at[j]
      zero_accum()
      plsc.subcore_barrier()

      load_group(0, 0, True)
      for b in range(NB):
        pltpu.async_copy(tbl.at[sbuf.at[b]], rows.at[b], sems.at[b])

      def one_group(g, par):
        nxt = 1 - par
        base = erow0 + (g + 1) * NB
        nsl = pl.ds(nxt * NB, NB)

        @pl.when(g > 0)
        def _():
          drain_scatters(nxt)

        @pl.when(g + 1 < NGRP)
        def _():
          pltpu.async_copy(src2.at[pl.ds(base, NB)], sbuf.at[nsl], isem)
          pltpu.async_copy(dst2.at[pl.ds(base, NB)], dbuf.at[nsl], isem)

        for b in range(NB):
          i = par * NB + b
          pltpu.make_async_copy(tbl.at[sbuf.at[i]], rows.at[i],
                                sems.at[b]).wait()
          pltpu.async_copy(rows.at[i], accum.at[dbuf.at[i]], ssems.at[i],
                           add=True)

        @pl.when(g + 1 < NGRP)
        def _():
          pltpu.make_async_copy(src2.at[pl.ds(base, NB)], sbuf.at[nsl],
                                isem).wait()
          pltpu.make_async_copy(dst2.at[pl.ds(base, NB)], dbuf.at[nsl],
                                isem).wait()
          for b in range(NB):
            ni = nxt * NB + b
            pltpu.async_copy(tbl.at[sbuf.at[ni]], rows.at[ni], sems.at[b])

      def body(i, carry):
        one_group(2 * i, 0)
        one_group(2 * i + 1, 1)
        return carry

      lax.fori_loop(0, NGRP // 2, body, 0)
      drain_scatters((NGRP - 1) % 2)
      plsc.subcore_barrier()
      flush(out.at[slice(None), j])

    if with_counts:
      def drain_ones(par, idx_buf):
        for b in range(NB):
          i = par * NB + b
          pltpu.make_async_copy(ones_v, accum.at[idx_buf.at[i]],
                                ssems.at[i]).wait()

      zero_accum()
      plsc.subcore_barrier()
      load_group(0, 0, False)

      def deg_group(g, par):
        nxt = 1 - par
        base = erow0 + (g + 1) * NB
        nsl = pl.ds(nxt * NB, NB)

        @pl.when(g > 0)
        def _():
          drain_ones(nxt, dbuf)

        @pl.when(g + 1 < NGRP)
        def _():
          pltpu.async_copy(dst2.at[pl.ds(base, NB)], dbuf.at[nsl], isem)

        for b in range(NB):
          i = par * NB + b
          pltpu.async_copy(ones_v, accum.at[dbuf.at[i]], ssems.at[i],
                           add=True)

        @pl.when(g + 1 < NGRP)
        def _():
          pltpu.make_async_copy(dst2.at[pl.ds(base, NB)], dbuf.at[nsl],
                                isem).wait()

      def deg_body(i, carry):
        deg_group(2 * i, 0)
        deg_group(2 * i + 1, 1)
        return carry

      lax.fori_loop(0, NGRP // 2, deg_body, 0)
      drain_ones((NGRP - 1) % 2, dbuf)
      plsc.subcore_barrier()
      flush(cnt_out.at[slice(None), 0])

      zero_accum()
      plsc.subcore_barrier()
      iota16 = lax.iota(jnp.int32, 16)
      trash = [N + (k * 16) % 256 + iota16 for k in range(WIN // 16)]
      load_group(0, 0, True)

      def self_group(g, par):
        nxt = 1 - par
        base = erow0 + (g + 1) * NB
        nsl = pl.ds(nxt * NB, NB)

        @pl.when(g > 0)
        def _():
          drain_ones(nxt, selbuf)

        @pl.when(g + 1 < NGRP)
        def _():
          pltpu.async_copy(src2.at[pl.ds(base, NB)], sbuf.at[nsl], isem)
          pltpu.async_copy(dst2.at[pl.ds(base, NB)], dbuf.at[nsl], isem)

        for b in range(NB):
          i = par * NB + b
          for k in range(WIN // 16):
            sv = sbuf[i, pl.ds(k * 16, 16)]
            dv = dbuf[i, pl.ds(k * 16, 16)]
            selbuf[i, pl.ds(k * 16, 16)] = jnp.where(sv == dv, dv, trash[k])
          pltpu.async_copy(ones_v, accum.at[selbuf.at[i]], ssems.at[i],
                           add=True)

        @pl.when(g + 1 < NGRP)
        def _():
          pltpu.make_async_copy(src2.at[pl.ds(base, NB)], sbuf.at[nsl],
                                isem).wait()
          pltpu.make_async_copy(dst2.at[pl.ds(base, NB)], dbuf.at[nsl],
                                isem).wait()

      def self_body(i, carry):
        self_group(2 * i, 0)
        self_group(2 * i + 1, 1)
        return carry

      lax.fori_loop(0, NGRP // 2, self_body, 0)
      drain_ones((NGRP - 1) % 2, selbuf)
      plsc.subcore_barrier()
      flush(cnt_out.at[slice(None), 1])

  return scatter_kernel


def _lrelu(v):
  return jnp.where(v >= 0.0, v, 0.1 * v)


def _mm(a, w):
  return jax.lax.dot_general(a, w, (((1,), (0,)), ((), ())),
                             precision=jax.lax.Precision.HIGHEST,
                             preferred_element_type=jnp.float32)


def _pk_spec(p):
  return pl.BlockSpec((p, BM, 128), lambda i: (0, i, 0))


_T_SPEC = lambda p: pl.BlockSpec((2, p, BM, 128), lambda i: (0, 0, i, 0))
_CNT_SPEC = pl.BlockSpec((2, 2, BM, 128), lambda i: (0, 0, i, 0))
_AB_SPEC = pl.BlockSpec((2, BM, 128), lambda i: (0, i, 0))


def _full_spec(shape):
  nd = len(shape)
  return pl.BlockSpec(shape, lambda i: (0,) * nd)


def _scalars_from_cnt(cnt):
  d = cnt[0, 0] + cnt[1, 0]
  s = cnt[0, 1] + cnt[1, 1]
  deg = 1.0 + d - s
  a = 1.0 / jnp.maximum(deg, 1.0)
  return a, a * (1.0 - s)


def _make_dense(p_in, p_out, form_b, p_next=None, first=False):
  def body(*refs):
    it = iter(refs)
    h_ref = next(it)
    g_ref = next(it) if form_b else None
    t_ref = next(it)
    sc_ref = next(it)
    wok_ref = None if form_b else next(it)
    bop_ref = next(it)
    wrk_ref = next(it)
    wnk_ref = next(it) if p_next is not None else None
    out_ref = next(it)
    gout_ref = next(it) if p_next is not None else None
    ab_ref = next(it) if first else None

    if first:
      a, b = _scalars_from_cnt(sc_ref[...])
      ab_ref[0] = a
      ab_ref[1] = b
    else:
      a = sc_ref[0]
      b = sc_ref[1]

    hs = [h_ref[i] for i in range(p_in)]
    vals = []
    for po in range(p_out):
      if form_b:
        val = a * (t_ref[0, po] + t_ref[1, po]) + b * g_ref[po]
        for pi in range(p_in):
          val = val + _mm(hs[pi], wrk_ref[pi, po])
      else:
        val = None
        for pi in range(p_in):
          agg = a * (t_ref[0, pi] + t_ref[1, pi]) + b * hs[pi]
          term = _mm(agg, wok_ref[pi, po]) + _mm(hs[pi], wrk_ref[pi, po])
          val = term if val is None else val + term
      val = _lrelu(val + bop_ref[po])
      vals.append(val)
      out_ref[po] = val
    if p_next is not None:
      for pn in range(p_next):
        gv = None
        for po in range(p_out):
          term = _mm(vals[po], wnk_ref[po, pn])
          gv = term if gv is None else gv + term
        gout_ref[pn] = gv

  in_specs = [_pk_spec(p_in)]
  if form_b:
    in_specs.append(_pk_spec(p_out))
  in_specs.append(_T_SPEC(p_out if form_b else p_in))
  in_specs.append(_CNT_SPEC if first else _AB_SPEC)
  if not form_b:
    in_specs.append(_full_spec((p_in, p_out, 128, 128)))
  in_specs.append(_full_spec((p_out, 128)))
  in_specs.append(_full_spec((p_in, p_out, 128, 128)))
  if p_next is not None:
    in_specs.append(_full_spec((p_out, p_next, 128, 128)))

  out_shape = [jax.ShapeDtypeStruct((p_out, M, 128), jnp.float32)]
  out_specs = [_pk_spec(p_out)]
  if p_next is not None:
    out_shape.append(jax.ShapeDtypeStruct((p_next, M, 128), jnp.float32))
    out_specs.append(_pk_spec(p_next))
  if first:
    out_shape.append(jax.ShapeDtypeStruct((2, MP, 128), jnp.float32))
    out_specs.append(_AB_SPEC)

  return pl.pallas_call(
      body,
      grid=(GRID,),
      in_specs=in_specs,
      out_specs=out_specs if len(out_specs) > 1 else out_specs[0],
      out_shape=out_shape if len(out_shape) > 1 else out_shape[0],
  )


def _xpack_body(x_ref, p_ref, o_ref):
  o_ref[0] = _mm(x_ref[...], p_ref[...])


_xpack = pl.pallas_call(
    _xpack_body,
    grid=(GRID,),
    in_specs=[pl.BlockSpec((BM, 24), lambda i: (i, 0)),
              _full_spec((24, 128))],
    out_specs=pl.BlockSpec((1, BM, 128), lambda i: (0, i, 0)),
    out_shape=jax.ShapeDtypeStruct((1, M, 128), jnp.float32),
)


def _unpack_body(v_ref, s_ref, o_ref):
  o_ref[...] = _mm(v_ref[0], s_ref[...])


_unpack = pl.pallas_call(
    _unpack_body,
    grid=(GRID,),
    in_specs=[pl.BlockSpec((1, BM, 128), lambda i: (0, i, 0)),
              _full_spec((128, 8))],
    out_specs=pl.BlockSpec((BM, 8), lambda i: (i, 0)),
    out_shape=jax.ShapeDtypeStruct((M, 8), jnp.float32),
)


_scatter1c = _make_scatter_kernel(1, True)
_scatter1 = _make_scatter_kernel(1, False)
_scatter2 = _make_scatter_kernel(2, False)
_scatter4 = _make_scatter_kernel(4, False)

_dense0 = _make_dense(1, 1, False, first=True)
_dense1 = _make_dense(1, 1, False)
_dense2 = _make_dense(1, 2, False)
_dense3 = _make_dense(2, 4, False)
_dense4 = _make_dense(4, 4, False, p_next=2)
_dense5 = _make_dense(4, 2, True, p_next=1)
_dense6 = _make_dense(2, 1, True)
_dense7 = _make_dense(1, 1, False)


def _pad16(w):
  di, do = w.shape
  return jnp.pad(w, ((0, (-di) % 16), (0, (-do) % 16)))


def _kron8(w):
  w = _pad16(w)
  pi, po = w.shape[0] // 16, w.shape[1] // 16
  i8 = jnp.eye(8, dtype=w.dtype)
  return jnp.stack([
      jnp.stack([jnp.kron(i8, w[16 * i:16 * i + 16, 16 * j:16 * j + 16])
                 for j in range(po)])
      for i in range(pi)])


def _packb(bo):
  bo = jnp.pad(bo, (0, (-bo.shape[0]) % 16))
  po = bo.shape[0] // 16
  return jnp.tile(bo.reshape(po, 1, 16), (1, 8, 1)).reshape(po, 128)


def _sc_view(hpk):
  p = hpk.shape[0]
  return hpk.reshape(p, M * 8, 16)


def kernel(x, edge_index,
           Wo0, bo0, Wr0, Wo1, bo1, Wr1, Wo2, bo2, Wr2, Wo3, bo3, Wr3,
           Wo4, bo4, Wr4, Wo5, bo5, Wr5, Wo6, bo6, Wr6, Wo7, bo7, Wr7):
  f32 = jnp.float32
  pad = jnp.stack([jnp.zeros((EP - E,), jnp.int32),
                   jnp.full((EP - E,), N, jnp.int32)])
  ei3 = jnp.concatenate([edge_index, pad], axis=1).reshape(2, EROWS, WIN)
  src = ei3[0]
  dst = ei3[1]
  zeros16 = jnp.zeros((WIN, 16), f32)
  ones16 = jnp.ones((WIN, 16), f32)

  q = jnp.arange(8).repeat(3)
  f = jnp.tile(jnp.arange(3), 8)
  perm = jnp.zeros((24, 128), f32).at[3 * q + f, 16 * q + f].set(1.0)
  xpk = _xpack(x.reshape(M, 24), perm)

  wok = [_kron8(w) for w in (Wo0, Wo1, Wo2, Wo3, Wo4, Wo5, Wo6, Wo7)]
  wrk = [_kron8(w) for w in (Wr0, Wr1, Wr2, Wr3, Wr4, Wr5, Wr6, Wr7)]
  bop = [_packb(b) for b in (bo0, bo1, bo2, bo3, bo4, bo5, bo6, bo7)]

  def t_view(t):
    return t.reshape(t.shape[0], t.shape[1], MP, 128)

  t0, cnt = _scatter1c(_sc_view(xpk), src, dst, zeros16, ones16)
  cntv = cnt.reshape(2, 2, MP, 128)
  h1, ab = _dense0(xpk, t_view(t0), cntv, wok[0], bop[0], wrk[0])
  t1 = _scatter1(_sc_view(h1), src, dst, zeros16, ones16)
  h2 = _dense1(h1, t_view(t1), ab, wok[1], bop[1], wrk[1])
  t2 = _scatter1(_sc_view(h2), src, dst, zeros16, ones16)
  h3 = _dense2(h2, t_view(t2), ab, wok[2], bop[2], wrk[2])
  t3 = _scatter2(_sc_view(h3), src, dst, zeros16, ones16)
  h4 = _dense3(h3, t_view(t3), ab, wok[3], bop[3], wrk[3])
  t4 = _scatter4(_sc_view(h4), src, dst, zeros16, ones16)
  h5, g5 = _dense4(h4, t_view(t4), ab, wok[4], bop[4], wrk[4], wok[5])
  t5 = _scatter2(_sc_view(g5), src, dst, zeros16, ones16)
  h6, g6 = _dense5(h5, g5, t_view(t5), ab, bop[5], wrk[5], wok[6])
  t6 = _scatter1(_sc_view(g6), src, dst, zeros16, ones16)
  h7 = _dense6(h6, g6, t_view(t6), ab, bop[6], wrk[6])
  t7 = _scatter1(_sc_view(h7), src, dst, zeros16, ones16)
  out8 = _dense7(h7, t_view(t7), ab, wok[7], bop[7], wrk[7])
  sel = jnp.zeros((128, 8), f32).at[jnp.arange(8) * 16, jnp.arange(8)].set(1.0)
  return _unpack(out8, sel).reshape(N)


# speedup vs baseline: 26.2510x; 1.0955x over previous
import functools

import jax
import jax.numpy as jnp
from jax import lax
from jax.experimental import pallas as pl
from jax.experimental.pallas import tpu as pltpu
from jax.experimental.pallas import tpu_sc as plsc

N = 100000
E = 1600000
WIN = 128
NB = 4
WPT = 392
NGRP = WPT // NB
NTILES = 32
EP = NTILES * WPT * WIN
EROWS = EP // WIN
ROWS_PT = 6272
NP = 16 * ROWS_PT
M = N // 8
MP = NP // 8
BM = 256
GRID = (M + BM - 1) // BM


def _make_scatter_kernel(p, with_counts):
  mesh = plsc.VectorSubcoreMesh(core_axis_name="c", subcore_axis_name="s",
                                num_cores=2, num_subcores=16)
  t_type = jax.ShapeDtypeStruct((2, p, NP, 16), jnp.float32)
  if with_counts:
    out_type = [t_type, jax.ShapeDtypeStruct((2, 2, NP, 16), jnp.float32)]
  else:
    out_type = t_type

  @functools.partial(
      pl.kernel,
      out_type=out_type,
      mesh=mesh,
      compiler_params=pltpu.CompilerParams(use_tc_tiling_on_sc=False),
      scratch_types=[
          pltpu.VMEM((2 * NB, WIN), jnp.int32),
          pltpu.VMEM((2 * NB, WIN), jnp.int32),
          pltpu.VMEM((2 * NB, WIN), jnp.int32),
          pltpu.VMEM((2 * NB, WIN, 16), jnp.float32),
          pltpu.VMEM((WIN, 16), jnp.float32),
          pltpu.VMEM((WIN, 16), jnp.float32),
          pltpu.VMEM_SHARED((NP, 16), jnp.float32),
          pltpu.SemaphoreType.DMA((NB,)),
          pltpu.SemaphoreType.DMA((2 * NB,)),
          pltpu.SemaphoreType.DMA,
      ],
  )
  def scatter_kernel(table, src2, dst2, zeros_hbm, ones_hbm, *rest):
    if with_counts:
      out, cnt_out = rest[0], rest[1]
      scratch = rest[2:]
    else:
      out = rest[0]
      cnt_out = None
      scratch = rest[1:]
    (sbuf, dbuf, selbuf, rows, zeros_v, ones_v, accum, sems, ssems,
     isem) = scratch
    c = lax.axis_index("c")
    s = lax.axis_index("s")
    erow0 = (c * 16 + s) * WPT
    r0 = s * ROWS_PT
    pltpu.sync_copy(zeros_hbm, zeros_v)
    if with_counts:
      pltpu.sync_copy(ones_hbm, ones_v)

    def zero_accum():
      for k in range(ROWS_PT // WIN):
        pltpu.sync_copy(zeros_v, accum.at[pl.ds(r0 + k * WIN, WIN)])

    def flush(dst_ref):
      sl = pl.ds(r0, ROWS_PT)
      pltpu.sync_copy(accum.at[sl], dst_ref.at[c, sl])

    def load_group(g, par, want_src):
      base = erow0 + g * NB
      if want_src:
        pltpu.sync_copy(src2.at[pl.ds(base, NB)],
                        sbuf.at[pl.ds(par * NB, NB)])
      pltpu.sync_copy(dst2.at[pl.ds(base, NB)],
                      dbuf.at[pl.ds(par * NB, NB)])

    def drain_scatters(par):
      for b in range(NB):
        i = par * NB + b
        pltpu.make_async_copy(rows.at[i], accum.at[dbuf.at[i]],
                              ssems.at[i]).wait()

    for j in range(p):
      tbl = table.at[j]
      zero_accum()
      plsc.subcore_barrier()

      load_group(0, 0, True)
      for b in range(NB):
        pltpu.async_copy(tbl.at[sbuf.at[b]], rows.at[b], sems.at[b])

      def one_group(g, par):
        nxt = 1 - par
        base = erow0 + (g + 1) * NB
        nsl = pl.ds(nxt * NB, NB)

        @pl.when(g > 0)
        def _():
          drain_scatters(nxt)

        @pl.when(g + 1 < NGRP)
        def _():
          pltpu.async_copy(src2.at[pl.ds(base, NB)], sbuf.at[nsl], isem)
          pltpu.async_copy(dst2.at[pl.ds(base, NB)], dbuf.at[nsl], isem)

        for b in range(NB):
          i = par * NB + b
          pltpu.make_async_copy(tbl.at[sbuf.at[i]], rows.at[i],
                                sems.at[b]).wait()
          pltpu.async_copy(rows.at[i], accum.at[dbuf.at[i]], ssems.at[i],
                           add=True)

        @pl.when(g + 1 < NGRP)
        def _():
          pltpu.make_async_copy(src2.at[pl.ds(base, NB)], sbuf.at[nsl],
                                isem).wait()
          pltpu.make_async_copy(dst2.at[pl.ds(base, NB)], dbuf.at[nsl],
                                isem).wait()
          for b in range(NB):
            ni = nxt * NB + b
            pltpu.async_copy(tbl.at[sbuf.at[ni]], rows.at[ni], sems.at[b])

      def body(i, carry):
        one_group(2 * i, 0)
        one_group(2 * i + 1, 1)
        return carry

      lax.fori_loop(0, NGRP // 2, body, 0)
      drain_scatters((NGRP - 1) % 2)
      plsc.subcore_barrier()
      flush(out.at[slice(None), j])

    if with_counts:
      def drain_ones(par, idx_buf):
        for b in range(NB):
          i = par * NB + b
          pltpu.make_async_copy(ones_v, accum.at[idx_buf.at[i]],
                                ssems.at[i]).wait()

      zero_accum()
      plsc.subcore_barrier()
      load_group(0, 0, False)

      def deg_group(g, par):
        nxt = 1 - par
        base = erow0 + (g + 1) * NB
        nsl = pl.ds(nxt * NB, NB)

        @pl.when(g > 0)
        def _():
          drain_ones(nxt, dbuf)

        @pl.when(g + 1 < NGRP)
        def _():
          pltpu.async_copy(dst2.at[pl.ds(base, NB)], dbuf.at[nsl], isem)

        for b in range(NB):
          i = par * NB + b
          pltpu.async_copy(ones_v, accum.at[dbuf.at[i]], ssems.at[i],
                           add=True)

        @pl.when(g + 1 < NGRP)
        def _():
          pltpu.make_async_copy(dst2.at[pl.ds(base, NB)], dbuf.at[nsl],
                                isem).wait()

      def deg_body(i, carry):
        deg_group(2 * i, 0)
        deg_group(2 * i + 1, 1)
        return carry

      lax.fori_loop(0, NGRP // 2, deg_body, 0)
      drain_ones((NGRP - 1) % 2, dbuf)
      plsc.subcore_barrier()
      flush(cnt_out.at[slice(None), 0])

      zero_accum()
      plsc.subcore_barrier()
      iota16 = lax.iota(jnp.int32, 16)
      trash = [N + (k * 16) % 256 + iota16 for k in range(WIN // 16)]
      load_group(0, 0, True)

      def self_group(g, par):
        nxt = 1 - par
        base = erow0 + (g + 1) * NB
        nsl = pl.ds(nxt * NB, NB)

        @pl.when(g > 0)
        def _():
          drain_ones(nxt, selbuf)

        @pl.when(g + 1 < NGRP)
        def _():
          pltpu.async_copy(src2.at[pl.ds(base, NB)], sbuf.at[nsl], isem)
          pltpu.async_copy(dst2.at[pl.ds(base, NB)], dbuf.at[nsl], isem)

        for b in range(NB):
          i = par * NB + b
          for k in range(WIN // 16):
            sv = sbuf[i, pl.ds(k * 16, 16)]
            dv = dbuf[i, pl.ds(k * 16, 16)]
            selbuf[i, pl.ds(k * 16, 16)] = jnp.where(sv == dv, dv, trash[k])
          pltpu.async_copy(ones_v, accum.at[selbuf.at[i]], ssems.at[i],
                           add=True)

        @pl.when(g + 1 < NGRP)
        def _():
          pltpu.make_async_copy(src2.at[pl.ds(base, NB)], sbuf.at[nsl],
                                isem).wait()
          pltpu.make_async_copy(dst2.at[pl.ds(base, NB)], dbuf.at[nsl],
                                isem).wait()

      def self_body(i, carry):
        self_group(2 * i, 0)
        self_group(2 * i + 1, 1)
        return carry

      lax.fori_loop(0, NGRP // 2, self_body, 0)
      drain_ones((NGRP - 1) % 2, selbuf)
      plsc.subcore_barrier()
      flush(cnt_out.at[slice(None), 1])

  return scatter_kernel


def _lrelu(v):
  return jnp.where(v >= 0.0, v, 0.1 * v)


def _mm(a, w):
  return jax.lax.dot_general(a, w, (((1,), (0,)), ((), ())),
                             preferred_element_type=jnp.float32)


def _pk_spec(p):
  return pl.BlockSpec((p, BM, 128), lambda i: (0, i, 0))


_T_SPEC = lambda p: pl.BlockSpec((2, p, BM, 128), lambda i: (0, 0, i, 0))
_CNT_SPEC = pl.BlockSpec((2, 2, BM, 128), lambda i: (0, 0, i, 0))
_AB_SPEC = pl.BlockSpec((2, BM, 128), lambda i: (0, i, 0))


def _full_spec(shape):
  nd = len(shape)
  return pl.BlockSpec(shape, lambda i: (0,) * nd)


def _scalars_from_cnt(cnt):
  d = cnt[0, 0] + cnt[1, 0]
  s = cnt[0, 1] + cnt[1, 1]
  deg = 1.0 + d - s
  a = 1.0 / jnp.maximum(deg, 1.0)
  return a, a * (1.0 - s)


def _make_dense(p_in, p_out, form_b, p_next=None, first=False):
  def body(*refs):
    it = iter(refs)
    h_ref = next(it)
    g_ref = next(it) if form_b else None
    t_ref = next(it)
    sc_ref = next(it)
    wok_ref = None if form_b else next(it)
    bop_ref = next(it)
    wrk_ref = next(it)
    wnk_ref = next(it) if p_next is not None else None
    out_ref = next(it)
    gout_ref = next(it) if p_next is not None else None
    ab_ref = next(it) if first else None

    if first:
      a, b = _scalars_from_cnt(sc_ref[...])
      ab_ref[0] = a
      ab_ref[1] = b
    else:
      a = sc_ref[0]
      b = sc_ref[1]

    hs = [h_ref[i] for i in range(p_in)]
    vals = []
    for po in range(p_out):
      if form_b:
        val = a * (t_ref[0, po] + t_ref[1, po]) + b * g_ref[po]
        for pi in range(p_in):
          val = val + _mm(hs[pi], wrk_ref[pi, po])
      else:
        val = None
        for pi in range(p_in):
          agg = a * (t_ref[0, pi] + t_ref[1, pi]) + b * hs[pi]
          term = _mm(agg, wok_ref[pi, po]) + _mm(hs[pi], wrk_ref[pi, po])
          val = term if val is None else val + term
      val = _lrelu(val + bop_ref[po])
      vals.append(val)
      out_ref[po] = val
    if p_next is not None:
      for pn in range(p_next):
        gv = None
        for po in range(p_out):
          term = _mm(vals[po], wnk_ref[po, pn])
          gv = term if gv is None else gv + term
        gout_ref[pn] = gv

  in_specs = [_pk_spec(p_in)]
  if form_b:
    in_specs.append(_pk_spec(p_out))
  in_specs.append(_T_SPEC(p_out if form_b else p_in))
  in_specs.append(_CNT_SPEC if first else _AB_SPEC)
  if not form_b:
    in_specs.append(_full_spec((p_in, p_out, 128, 128)))
  in_specs.append(_full_spec((p_out, 128)))
  in_specs.append(_full_spec((p_in, p_out, 128, 128)))
  if p_next is not None:
    in_specs.append(_full_spec((p_out, p_next, 128, 128)))

  out_shape = [jax.ShapeDtypeStruct((p_out, M, 128), jnp.float32)]
  out_specs = [_pk_spec(p_out)]
  if p_next is not None:
    out_shape.append(jax.ShapeDtypeStruct((p_next, M, 128), jnp.float32))
    out_specs.append(_pk_spec(p_next))
  if first:
    out_shape.append(jax.ShapeDtypeStruct((2, MP, 128), jnp.float32))
    out_specs.append(_AB_SPEC)

  return pl.pallas_call(
      body,
      grid=(GRID,),
      in_specs=in_specs,
      out_specs=out_specs if len(out_specs) > 1 else out_specs[0],
      out_shape=out_shape if len(out_shape) > 1 else out_shape[0],
  )


def _xpack_body(x_ref, p_ref, o_ref):
  o_ref[0] = _mm(x_ref[...], p_ref[...])


_xpack = pl.pallas_call(
    _xpack_body,
    grid=(GRID,),
    in_specs=[pl.BlockSpec((BM, 24), lambda i: (i, 0)),
              _full_spec((24, 128))],
    out_specs=pl.BlockSpec((1, BM, 128), lambda i: (0, i, 0)),
    out_shape=jax.ShapeDtypeStruct((1, M, 128), jnp.float32),
)


def _unpack_body(v_ref, s_ref, o_ref):
  o_ref[...] = _mm(v_ref[0], s_ref[...])


_unpack = pl.pallas_call(
    _unpack_body,
    grid=(GRID,),
    in_specs=[pl.BlockSpec((1, BM, 128), lambda i: (0, i, 0)),
              _full_spec((128, 8))],
    out_specs=pl.BlockSpec((BM, 8), lambda i: (i, 0)),
    out_shape=jax.ShapeDtypeStruct((M, 8), jnp.float32),
)


_scatter1c = _make_scatter_kernel(1, True)
_scatter1 = _make_scatter_kernel(1, False)
_scatter2 = _make_scatter_kernel(2, False)
_scatter4 = _make_scatter_kernel(4, False)

_dense0 = _make_dense(1, 1, False, first=True)
_dense1 = _make_dense(1, 1, False)
_dense2 = _make_dense(1, 2, False)
_dense3 = _make_dense(2, 4, False)
_dense4 = _make_dense(4, 4, False, p_next=2)
_dense5 = _make_dense(4, 2, True, p_next=1)
_dense6 = _make_dense(2, 1, True)
_dense7 = _make_dense(1, 1, False)


def _pad16(w):
  di, do = w.shape
  return jnp.pad(w, ((0, (-di) % 16), (0, (-do) % 16)))


def _kron8(w):
  w = _pad16(w)
  pi, po = w.shape[0] // 16, w.shape[1] // 16
  i8 = jnp.eye(8, dtype=w.dtype)
  return jnp.stack([
      jnp.stack([jnp.kron(i8, w[16 * i:16 * i + 16, 16 * j:16 * j + 16])
                 for j in range(po)])
      for i in range(pi)])


def _packb(bo):
  bo = jnp.pad(bo, (0, (-bo.shape[0]) % 16))
  po = bo.shape[0] // 16
  return jnp.tile(bo.reshape(po, 1, 16), (1, 8, 1)).reshape(po, 128)


def _sc_view(hpk):
  p = hpk.shape[0]
  return hpk.reshape(p, M * 8, 16)


def kernel(x, edge_index,
           Wo0, bo0, Wr0, Wo1, bo1, Wr1, Wo2, bo2, Wr2, Wo3, bo3, Wr3,
           Wo4, bo4, Wr4, Wo5, bo5, Wr5, Wo6, bo6, Wr6, Wo7, bo7, Wr7):
  f32 = jnp.float32
  pad = jnp.stack([jnp.zeros((EP - E,), jnp.int32),
                   jnp.full((EP - E,), N, jnp.int32)])
  ei3 = jnp.concatenate([edge_index, pad], axis=1).reshape(2, EROWS, WIN)
  src = ei3[0]
  dst = ei3[1]
  zeros16 = jnp.zeros((WIN, 16), f32)
  ones16 = jnp.ones((WIN, 16), f32)

  q = jnp.arange(8).repeat(3)
  f = jnp.tile(jnp.arange(3), 8)
  perm = jnp.zeros((24, 128), f32).at[3 * q + f, 16 * q + f].set(1.0)
  xpk = _xpack(x.reshape(M, 24), perm)

  wok = [_kron8(w) for w in (Wo0, Wo1, Wo2, Wo3, Wo4, Wo5, Wo6, Wo7)]
  wrk = [_kron8(w) for w in (Wr0, Wr1, Wr2, Wr3, Wr4, Wr5, Wr6, Wr7)]
  bop = [_packb(b) for b in (bo0, bo1, bo2, bo3, bo4, bo5, bo6, bo7)]

  def t_view(t):
    return t.reshape(t.shape[0], t.shape[1], MP, 128)

  t0, cnt = _scatter1c(_sc_view(xpk), src, dst, zeros16, ones16)
  cntv = cnt.reshape(2, 2, MP, 128)
  h1, ab = _dense0(xpk, t_view(t0), cntv, wok[0], bop[0], wrk[0])
  t1 = _scatter1(_sc_view(h1), src, dst, zeros16, ones16)
  h2 = _dense1(h1, t_view(t1), ab, wok[1], bop[1], wrk[1])
  t2 = _scatter1(_sc_view(h2), src, dst, zeros16, ones16)
  h3 = _dense2(h2, t_view(t2), ab, wok[2], bop[2], wrk[2])
  t3 = _scatter2(_sc_view(h3), src, dst, zeros16, ones16)
  h4 = _dense3(h3, t_view(t3), ab, wok[3], bop[3], wrk[3])
  t4 = _scatter4(_sc_view(h4), src, dst, zeros16, ones16)
  h5, g5 = _dense4(h4, t_view(t4), ab, wok[4], bop[4], wrk[4], wok[5])
  t5 = _scatter2(_sc_view(g5), src, dst, zeros16, ones16)
  h6, g6 = _dense5(h5, g5, t_view(t5), ab, bop[5], wrk[5], wok[6])
  t6 = _scatter1(_sc_view(g6), src, dst, zeros16, ones16)
  h7 = _dense6(h6, g6, t_view(t6), ab, bop[6], wrk[6])
  t7 = _scatter1(_sc_view(h7), src, dst, zeros16, ones16)
  out8 = _dense7(h7, t_view(t7), ab, wok[7], bop[7], wrk[7])
  sel = jnp.zeros((128, 8), f32).at[jnp.arange(8) * 16, jnp.arange(8)].set(1.0)
  return _unpack(out8, sel).reshape(N)
